# Initial kernel scaffold; baseline (speedup 1.0000x reference)
import numpy as np
import jax
import jax.numpy as jnp
from jax import lax
from jax.experimental import pallas as pl
from jax.experimental.pallas import tpu as pltpu
from jax.experimental.pallas import tpu_sc as plsc

N = 10000
E = 320000
D = 128
H = 8
CH = 8
HC = H * CH
NCLS = 64
SLOPE = 0.2
EPS = 1e-16

NC = 2
NS = 16
NW = NC * NS
EPW = E // NW
CB = 400
NCHUNK = EPW // CB
SUB = 5
SUBB = CB // SUB
ROWS_PER_TILE = N // NS

_MASK8 = np.zeros((HC, H), np.float32)
for _h in range(H):
    _MASK8[_h * CH:(_h + 1) * CH, _h] = 1.0


def _iota16():
    return lax.iota(jnp.int32, 16)


def _splat(v):
    return jnp.full((16,), v, jnp.int32)


def _leaky_exp(e):
    return jnp.exp(jnp.where(e > 0, e, SLOPE * e))


def _sc1_body(src_hbm, dst_hbm, as_hbm, ad_hbm, z8_hbm,
              den_out, ex_out,
              sidx, didx2, asr, adr, exc, accum, sem):
    cid = lax.axis_index("c")
    sid = lax.axis_index("s")
    wid = sid * NC + cid
    base = wid * EPW

    lane = _iota16()
    half = lane >> 3
    mod8 = lane & 7

    pltpu.sync_copy(z8_hbm.at[pl.ds(sid * ROWS_PER_TILE, ROWS_PER_TILE)],
                    accum.at[pl.ds(sid * ROWS_PER_TILE, ROWS_PER_TILE)])
    plsc.subcore_barrier()

    def chunk(c, _):
        off = base + c * CB
        pltpu.sync_copy(src_hbm.at[pl.ds(off, CB)], sidx)
        for j in range(SUB):
            pltpu.sync_copy(dst_hbm.at[pl.ds(off + j * SUBB, SUBB)],
                            didx2.at[j])
        cps = []
        for j in range(SUB):
            isl = pl.ds(j * SUBB, SUBB)
            cps.append(pltpu.async_copy(as_hbm.at[sidx.at[isl]],
                                        asr.at[isl, :], sem))
            cps.append(pltpu.async_copy(ad_hbm.at[didx2.at[j]],
                                        adr.at[isl, :], sem))
        for cp in cps:
            cp.wait()

        def vec(k, _):
            row = _splat(2 * k) + half
            e = (plsc.load_gather(asr, [row, mod8])
                 + plsc.load_gather(adr, [row, mod8]))
            plsc.store_scatter(exc, [row, mod8], _leaky_exp(e))
            return 0
        lax.fori_loop(0, CB * H // 16, vec, 0)

        scps = []
        for j in range(SUB):
            scps.append(pltpu.async_copy(exc.at[pl.ds(j * SUBB, SUBB), :],
                                         accum.at[didx2.at[j]], sem, add=True))
        for cp in scps:
            cp.wait()
        pltpu.sync_copy(exc, ex_out.at[pl.ds(off, CB), :])
        return 0

    lax.fori_loop(0, NCHUNK, chunk, 0)
    plsc.subcore_barrier()

    @pl.when(sid == 0)
    def _():
        pltpu.sync_copy(accum, den_out.at[cid])


_sc1 = pl.kernel(
    _sc1_body,
    out_type=(jax.ShapeDtypeStruct((NC, N, H), jnp.float32),
              jax.ShapeDtypeStruct((E, H), jnp.float32)),
    mesh=plsc.VectorSubcoreMesh(core_axis_name="c", subcore_axis_name="s"),
    scratch_types=[
        pltpu.VMEM((CB,), jnp.int32),
        pltpu.VMEM((SUB, SUBB), jnp.int32),
        pltpu.VMEM((CB, H), jnp.float32),
        pltpu.VMEM((CB, H), jnp.float32),
        pltpu.VMEM((CB, H), jnp.float32),
        pltpu.VMEM_SHARED((N, H), jnp.float32),
        pltpu.SemaphoreType.DMA,
    ],
)


def _sc2_body(src_hbm, dst_hbm, ex_hbm, d0_hbm, d1_hbm, h_hbm, z64_hbm,
              osum_out,
              sidx, didx2, exc, d0r, d1r, alc, hr, msg, accum, sem):
    cid = lax.axis_index("c")
    sid = lax.axis_index("s")
    wid = sid * NC + cid
    base = wid * EPW

    lane = _iota16()
    half = lane >> 3
    mod8 = lane & 7

    pltpu.sync_copy(z64_hbm.at[pl.ds(sid * ROWS_PER_TILE, ROWS_PER_TILE)],
                    accum.at[pl.ds(sid * ROWS_PER_TILE, ROWS_PER_TILE)])
    plsc.subcore_barrier()

    def chunk(c, _):
        off = base + c * CB
        pltpu.sync_copy(src_hbm.at[pl.ds(off, CB)], sidx)
        for j in range(SUB):
            pltpu.sync_copy(dst_hbm.at[pl.ds(off + j * SUBB, SUBB)],
                            didx2.at[j])
        cps = [pltpu.async_copy(ex_hbm.at[pl.ds(off, CB), :], exc, sem)]
        for j in range(SUB):
            isl = pl.ds(j * SUBB, SUBB)
            cps.append(pltpu.async_copy(d0_hbm.at[didx2.at[j]],
                                        d0r.at[isl, :], sem))
            cps.append(pltpu.async_copy(d1_hbm.at[didx2.at[j]],
                                        d1r.at[isl, :], sem))
            cps.append(pltpu.async_copy(h_hbm.at[sidx.at[isl]],
                                        hr.at[isl, :], sem))
        for cp in cps:
            cp.wait()

        def veca(k, _):
            row = _splat(2 * k) + half
            den = (plsc.load_gather(d0r, [row, mod8])
                   + plsc.load_gather(d1r, [row, mod8]))
            ex = plsc.load_gather(exc, [row, mod8])
            alc[pl.ds(16 * k, 16)] = ex / (den + EPS)
            return 0
        lax.fori_loop(0, CB * H // 16, veca, 0)

        def vecb(e, _):
            erow = _splat(e)
            for j in range(4):
                col = _splat(16 * j) + lane
                av = plsc.load_gather(alc, [_splat(H * e + 2 * j) + half])
                hv = plsc.load_gather(hr, [erow, col])
                plsc.store_scatter(msg, [erow, col], hv * av)
            return 0
        lax.fori_loop(0, CB, vecb, 0)

        scps = []
        for j in range(SUB):
            scps.append(pltpu.async_copy(msg.at[pl.ds(j * SUBB, SUBB), :],
                                         accum.at[didx2.at[j]], sem, add=True))
        for cp in scps:
            cp.wait()
        return 0

    lax.fori_loop(0, NCHUNK, chunk, 0)
    plsc.subcore_barrier()

    @pl.when(sid == 0)
    def _():
        pltpu.sync_copy(accum, osum_out.at[cid])


_sc2 = pl.kernel(
    _sc2_body,
    out_type=jax.ShapeDtypeStruct((NC, N, HC), jnp.float32),
    mesh=plsc.VectorSubcoreMesh(core_axis_name="c", subcore_axis_name="s"),
    scratch_types=[
        pltpu.VMEM((CB,), jnp.int32),
        pltpu.VMEM((SUB, SUBB), jnp.int32),
        pltpu.VMEM((CB, H), jnp.float32),
        pltpu.VMEM((CB, H), jnp.float32),
        pltpu.VMEM((CB, H), jnp.float32),
        pltpu.VMEM((CB * H,), jnp.float32),
        pltpu.VMEM((CB, HC), jnp.float32),
        pltpu.VMEM((CB, HC), jnp.float32),
        pltpu.VMEM_SHARED((N, HC), jnp.float32),
        pltpu.SemaphoreType.DMA,
    ],
)


def _sc3_body(src_hbm, dst_hbm, as_hbm, ad_hbm,
              den_out, ex_out,
              sidx, didx, as2t, ad2t, exc, dden, accum, sem):
    cid = lax.axis_index("c")
    sid = lax.axis_index("s")
    wid = sid * NC + cid
    base = wid * EPW

    pltpu.sync_copy(as_hbm, as2t)
    pltpu.sync_copy(ad_hbm, ad2t)

    def zero(i, _):
        dden[pl.ds(16 * i, 16)] = jnp.zeros((16,), jnp.float32)
        return 0
    lax.fori_loop(0, N // 16, zero, 0)

    @pl.when(sid == 0)
    def _():
        pltpu.sync_copy(dden, accum)
    plsc.subcore_barrier()

    def chunk(c, _):
        off = base + c * CB
        pltpu.sync_copy(src_hbm.at[pl.ds(off, CB)], sidx)
        pltpu.sync_copy(dst_hbm.at[pl.ds(off, CB)], didx)

        def vec(k, _):
            sl = pl.ds(16 * k, 16)
            sv = sidx[sl]
            dv = didx[sl]
            ex = _leaky_exp(plsc.load_gather(as2t, [sv])
                            + plsc.load_gather(ad2t, [dv]))
            exc[sl] = ex
            plsc.addupdate_scatter(dden, [dv], ex)
            return 0
        lax.fori_loop(0, CB // 16, vec, 0)

        pltpu.sync_copy(exc, ex_out.at[pl.ds(off, CB)])
        return 0

    lax.fori_loop(0, NCHUNK, chunk, 0)
    pltpu.sync_copy(dden, accum, add=True)
    plsc.subcore_barrier()

    @pl.when(sid == 0)
    def _():
        pltpu.sync_copy(accum, den_out.at[cid])


_sc3 = pl.kernel(
    _sc3_body,
    out_type=(jax.ShapeDtypeStruct((NC, N), jnp.float32),
              jax.ShapeDtypeStruct((E,), jnp.float32)),
    mesh=plsc.VectorSubcoreMesh(core_axis_name="c", subcore_axis_name="s"),
    scratch_types=[
        pltpu.VMEM((CB,), jnp.int32),
        pltpu.VMEM((CB,), jnp.int32),
        pltpu.VMEM((N,), jnp.float32),
        pltpu.VMEM((N,), jnp.float32),
        pltpu.VMEM((CB,), jnp.float32),
        pltpu.VMEM((N,), jnp.float32),
        pltpu.VMEM_SHARED((N,), jnp.float32),
        pltpu.SemaphoreType.DMA,
    ],
)


def _sc4_body(src_hbm, dst_hbm, ex_hbm, den_hbm, h_hbm, z64_hbm,
              osum_out,
              sidx, didx, didx2, d2t, exc, alc, hr, msg, accum, sem):
    cid = lax.axis_index("c")
    sid = lax.axis_index("s")
    wid = sid * NC + cid
    base = wid * EPW

    lane = _iota16()

    pltpu.sync_copy(den_hbm, d2t)
    pltpu.sync_copy(z64_hbm.at[pl.ds(sid * ROWS_PER_TILE, ROWS_PER_TILE)],
                    accum.at[pl.ds(sid * ROWS_PER_TILE, ROWS_PER_TILE)])
    plsc.subcore_barrier()

    def chunk(c, _):
        off = base + c * CB
        pltpu.sync_copy(src_hbm.at[pl.ds(off, CB)], sidx)
        pltpu.sync_copy(dst_hbm.at[pl.ds(off, CB)], didx)
        for j in range(SUB):
            pltpu.sync_copy(dst_hbm.at[pl.ds(off + j * SUBB, SUBB)],
                            didx2.at[j])
        cps = [pltpu.async_copy(ex_hbm.at[pl.ds(off, CB)], exc, sem)]
        for j in range(SUB):
            isl = pl.ds(j * SUBB, SUBB)
            cps.append(pltpu.async_copy(h_hbm.at[sidx.at[isl]],
                                        hr.at[isl, :], sem))
        for cp in cps:
            cp.wait()

        zero16 = _splat(0)
        one16 = _splat(1)

        def veca(k, _):
            sl = pl.ds(16 * k, 16)
            dv = didx[sl]
            den = (plsc.load_gather(d2t, [zero16, dv])
                   + plsc.load_gather(d2t, [one16, dv]))
            alc[sl] = exc[sl] / (den + EPS)
            return 0
        lax.fori_loop(0, CB // 16, veca, 0)

        def vecb(e, _):
            erow = _splat(e)
            av = plsc.load_gather(alc, [erow])
            for j in range(4):
                col = _splat(16 * j) + lane
                hv = plsc.load_gather(hr, [erow, col])
                plsc.store_scatter(msg, [erow, col], hv * av)
            return 0
        lax.fori_loop(0, CB, vecb, 0)

        scps = []
        for j in range(SUB):
            scps.append(pltpu.async_copy(msg.at[pl.ds(j * SUBB, SUBB), :],
                                         accum.at[didx2.at[j]], sem, add=True))
        for cp in scps:
            cp.wait()
        return 0

    lax.fori_loop(0, NCHUNK, chunk, 0)
    plsc.subcore_barrier()

    @pl.when(sid == 0)
    def _():
        pltpu.sync_copy(accum, osum_out.at[cid])


_sc4 = pl.kernel(
    _sc4_body,
    out_type=jax.ShapeDtypeStruct((NC, N, NCLS), jnp.float32),
    mesh=plsc.VectorSubcoreMesh(core_axis_name="c", subcore_axis_name="s"),
    scratch_types=[
        pltpu.VMEM((CB,), jnp.int32),
        pltpu.VMEM((CB,), jnp.int32),
        pltpu.VMEM((SUB, SUBB), jnp.int32),
        pltpu.VMEM((NC, N), jnp.float32),
        pltpu.VMEM((CB,), jnp.float32),
        pltpu.VMEM((CB,), jnp.float32),
        pltpu.VMEM((CB, NCLS), jnp.float32),
        pltpu.VMEM((CB, NCLS), jnp.float32),
        pltpu.VMEM_SHARED((N, NCLS), jnp.float32),
        pltpu.SemaphoreType.DMA,
    ],
)


_RB = 1000


def _tc1_body(x_ref, w_ref, a_ref, h_ref, al_ref):
    h = lax.dot_general(x_ref[...], w_ref[...], (((1,), (0,)), ((), ())),
                        preferred_element_type=jnp.float32)
    h_ref[...] = h
    al_ref[...] = lax.dot_general(h, a_ref[...], (((1,), (0,)), ((), ())),
                                  preferred_element_type=jnp.float32)


_tc1 = pl.pallas_call(
    _tc1_body,
    grid=(N // _RB,),
    in_specs=[pl.BlockSpec((_RB, D), lambda i: (i, 0)),
              pl.BlockSpec((D, HC), lambda i: (0, 0)),
              pl.BlockSpec((HC, 2 * H), lambda i: (0, 0))],
    out_specs=[pl.BlockSpec((_RB, HC), lambda i: (i, 0)),
               pl.BlockSpec((_RB, 2 * H), lambda i: (i, 0))],
    out_shape=[jax.ShapeDtypeStruct((N, HC), jnp.float32),
               jax.ShapeDtypeStruct((N, 2 * H), jnp.float32)],
)


def _tc2_body(o0_ref, o1_ref, b_ref, w_ref, a_ref, h_ref, al_ref):
    g = o0_ref[...] + o1_ref[...] + b_ref[...]
    g = jnp.where(g > 0, g, jnp.exp(g) - 1.0)
    h = lax.dot_general(g, w_ref[...], (((1,), (0,)), ((), ())),
                        preferred_element_type=jnp.float32)
    h_ref[...] = h
    al_ref[...] = lax.dot_general(h, a_ref[...], (((1,), (0,)), ((), ())),
                                  preferred_element_type=jnp.float32)


_tc2 = pl.pallas_call(
    _tc2_body,
    grid=(N // _RB,),
    in_specs=[pl.BlockSpec((_RB, HC), lambda i: (i, 0)),
              pl.BlockSpec((_RB, HC), lambda i: (i, 0)),
              pl.BlockSpec((1, HC), lambda i: (0, 0)),
              pl.BlockSpec((HC, NCLS), lambda i: (0, 0)),
              pl.BlockSpec((NCLS, 8), lambda i: (0, 0))],
    out_specs=[pl.BlockSpec((_RB, NCLS), lambda i: (i, 0)),
               pl.BlockSpec((_RB, 8), lambda i: (i, 0))],
    out_shape=[jax.ShapeDtypeStruct((N, NCLS), jnp.float32),
               jax.ShapeDtypeStruct((N, 8), jnp.float32)],
)


def _tc3_body(s0_ref, s1_ref, b_ref, o_ref):
    o_ref[...] = s0_ref[...] + s1_ref[...] + b_ref[...]


_tc3 = pl.pallas_call(
    _tc3_body,
    grid=(N // _RB,),
    in_specs=[pl.BlockSpec((_RB, NCLS), lambda i: (i, 0)),
              pl.BlockSpec((_RB, NCLS), lambda i: (i, 0)),
              pl.BlockSpec((1, NCLS), lambda i: (0, 0))],
    out_specs=pl.BlockSpec((_RB, NCLS), lambda i: (i, 0)),
    out_shape=jax.ShapeDtypeStruct((N, NCLS), jnp.float32),
)


def kernel(x, adjs, W1, a_src1, a_dst1, b1, W2, a_src2, a_dst2, b2):
    adjs = adjs.astype(jnp.int32)
    src = adjs[0]
    dst = adjs[1]

    mask8 = jnp.asarray(_MASK8)
    A1 = jnp.concatenate([mask8 * a_src1.reshape(HC, 1),
                          mask8 * a_dst1.reshape(HC, 1)], axis=1)
    A2 = jnp.concatenate([a_src2.reshape(NCLS, 1), a_dst2.reshape(NCLS, 1),
                          jnp.zeros((NCLS, 6), jnp.float32)], axis=1)

    z8 = jnp.zeros((N, H), jnp.float32)
    z64 = jnp.zeros((N, NCLS), jnp.float32)

    h1, al1 = _tc1(x, W1, A1)
    as1 = al1[:, :H]
    ad1 = al1[:, H:]

    dens, ex1 = _sc1(src, dst, as1, ad1, z8)
    osum = _sc2(src, dst, ex1, dens[0], dens[1], h1, z64)

    h2, al2 = _tc2(osum[0], osum[1], b1.reshape(1, HC), W2, A2)
    as2 = al2[:, 0]
    ad2 = al2[:, 1]

    den2, ex2 = _sc3(src, dst, as2, ad2)
    osum2 = _sc4(src, dst, ex2, den2, h2, z64)

    return _tc3(osum2[0], osum2[1], b2.reshape(1, NCLS))


# speedup vs baseline: 30.2779x; 30.2779x over previous
import numpy as np
import jax
import jax.numpy as jnp
from jax import lax
from jax.experimental import pallas as pl
from jax.experimental.pallas import tpu as pltpu
from jax.experimental.pallas import tpu_sc as plsc

N = 10000
E = 320000
D = 128
H = 8
CH = 8
HC = H * CH
NCLS = 64
SLOPE = 0.2
EPS = 1e-16

NC = 2
NS = 16
NW = NC * NS
EPW = E // NW
CB = 400
NCHUNK = EPW // CB
SUB = 5
SUBB = CB // SUB

_MASK8 = np.zeros((HC, H), np.float32)
for _h in range(H):
    _MASK8[_h * CH:(_h + 1) * CH, _h] = 1.0


def _iota16():
    return lax.iota(jnp.int32, 16)


def _splat(v):
    return jnp.full((16,), v, jnp.int32)


def _leaky_exp(e):
    return jnp.exp(jnp.where(e > 0, e, SLOPE * e))


_SC_PARAMS = pltpu.CompilerParams(needs_layout_passes=False,
                                  use_tc_tiling_on_sc=False)


def _att_body(src_hbm, dst_hbm, as_hbm, ad_hbm, z8_hbm,
              den_out, ex_out,
              sidx2, didx2, asr, adr, exc, accum, sem):
    cid = lax.axis_index("c")
    sid = lax.axis_index("s")
    wid = sid * NC + cid
    base = wid * EPW

    @pl.when(sid == 0)
    def _():
        pltpu.sync_copy(z8_hbm, accum)
    plsc.subcore_barrier()

    def chunk(c, _):
        off = base + c * CB
        for j in range(SUB):
            pltpu.sync_copy(src_hbm.at[pl.ds(off + j * SUBB, SUBB)],
                            sidx2.at[j])
            pltpu.sync_copy(dst_hbm.at[pl.ds(off + j * SUBB, SUBB)],
                            didx2.at[j])
        cps = []
        for j in range(SUB):
            isl = pl.ds(j * SUBB, SUBB)
            cps.append(pltpu.async_copy(as_hbm.at[sidx2.at[j]],
                                        asr.at[isl, :], sem))
            cps.append(pltpu.async_copy(ad_hbm.at[didx2.at[j]],
                                        adr.at[isl, :], sem))
        for cp in cps:
            cp.wait()

        lane = _iota16()
        half = lane >> 3
        mod8 = lane & 7

        def vec(k, _):
            row = _splat(2 * k) + half
            e = (plsc.load_gather(asr, [row, mod8])
                 + plsc.load_gather(adr, [row, mod8]))
            plsc.store_scatter(exc, [row, mod8], _leaky_exp(e))
            return 0
        lax.fori_loop(0, CB * H // 16, vec, 0)

        scps = []
        for j in range(SUB):
            scps.append(pltpu.async_copy(
                exc.at[pl.ds(j * SUBB, SUBB), :],
                accum.at[didx2.at[j]], sem, add=True))
        for cp in scps:
            cp.wait()
        pltpu.sync_copy(exc, ex_out.at[pl.ds(off, CB), :])
        return 0

    lax.fori_loop(0, NCHUNK, chunk, 0)
    plsc.subcore_barrier()

    @pl.when(sid == 0)
    def _():
        pltpu.sync_copy(accum, den_out.at[cid])


_att = pl.kernel(
    _att_body,
    out_type=(jax.ShapeDtypeStruct((NC, N, H), jnp.float32),
              jax.ShapeDtypeStruct((E, H), jnp.float32)),
    mesh=plsc.VectorSubcoreMesh(core_axis_name="c", subcore_axis_name="s"),
    scratch_types=[
        pltpu.VMEM((SUB, SUBB), jnp.int32),
        pltpu.VMEM((SUB, SUBB), jnp.int32),
        pltpu.VMEM((CB, H), jnp.float32),
        pltpu.VMEM((CB, H), jnp.float32),
        pltpu.VMEM((CB, H), jnp.float32),
        pltpu.VMEM_SHARED((N, H), jnp.float32),
        pltpu.SemaphoreType.DMA,
    ],
    compiler_params=_SC_PARAMS,
)


def _mk_msg(broadcast0):
    def body(src_hbm, dst_hbm, ex_hbm, d0_hbm, d1_hbm, h_hbm, z64_hbm,
             osum_out,
             sidx2, didx2, exc, d0r, d1r, alc, hr, msg, accum, sem):
        cid = lax.axis_index("c")
        sid = lax.axis_index("s")
        wid = sid * NC + cid
        base = wid * EPW

        lane = _iota16()
        half = lane >> 3
        mod8 = lane & 7

        @pl.when(sid == 0)
        def _():
            pltpu.sync_copy(z64_hbm, accum)
        plsc.subcore_barrier()

        def chunk(c, _):
            off = base + c * CB
            for j in range(SUB):
                pltpu.sync_copy(src_hbm.at[pl.ds(off + j * SUBB, SUBB)],
                                sidx2.at[j])
                pltpu.sync_copy(dst_hbm.at[pl.ds(off + j * SUBB, SUBB)],
                                didx2.at[j])
            cps = [pltpu.async_copy(ex_hbm.at[pl.ds(off, CB), :], exc, sem)]
            for j in range(SUB):
                isl = pl.ds(j * SUBB, SUBB)
                cps.append(pltpu.async_copy(d0_hbm.at[didx2.at[j]],
                                            d0r.at[isl, :], sem))
                cps.append(pltpu.async_copy(d1_hbm.at[didx2.at[j]],
                                            d1r.at[isl, :], sem))
                cps.append(pltpu.async_copy(h_hbm.at[sidx2.at[j]],
                                            hr.at[isl, :], sem))
            for cp in cps:
                cp.wait()

            def veca(k, _):
                row = _splat(2 * k) + half
                den = (plsc.load_gather(d0r, [row, mod8])
                       + plsc.load_gather(d1r, [row, mod8]))
                ex = plsc.load_gather(exc, [row, mod8])
                alc[pl.ds(16 * k, 16)] = ex / (den + EPS)
                return 0
            lax.fori_loop(0, CB * H // 16, veca, 0)

            def vecb(e, _):
                erow = _splat(e)
                for j in range(4):
                    col = _splat(16 * j) + lane
                    if broadcast0:
                        aidx = _splat(H * e)
                    else:
                        aidx = _splat(H * e + 2 * j) + half
                    av = plsc.load_gather(alc, [aidx])
                    hv = plsc.load_gather(hr, [erow, col])
                    plsc.store_scatter(msg, [erow, col], hv * av)
                return 0
            lax.fori_loop(0, CB, vecb, 0)

            scps = []
            for j in range(SUB):
                scps.append(pltpu.async_copy(
                    msg.at[pl.ds(j * SUBB, SUBB), :],
                    accum.at[didx2.at[j]], sem, add=True))
            for cp in scps:
                cp.wait()
            return 0

        lax.fori_loop(0, NCHUNK, chunk, 0)
        plsc.subcore_barrier()

        @pl.when(sid == 0)
        def _():
            pltpu.sync_copy(accum, osum_out.at[cid])

    return pl.kernel(
        body,
        out_type=jax.ShapeDtypeStruct((NC, N, HC), jnp.float32),
        mesh=plsc.VectorSubcoreMesh(core_axis_name="c", subcore_axis_name="s"),
        scratch_types=[
            pltpu.VMEM((SUB, SUBB), jnp.int32),
            pltpu.VMEM((SUB, SUBB), jnp.int32),
            pltpu.VMEM((CB, H), jnp.float32),
            pltpu.VMEM((CB, H), jnp.float32),
            pltpu.VMEM((CB, H), jnp.float32),
            pltpu.VMEM((CB * H,), jnp.float32),
            pltpu.VMEM((CB, HC), jnp.float32),
            pltpu.VMEM((CB, HC), jnp.float32),
            pltpu.VMEM_SHARED((N, HC), jnp.float32),
            pltpu.SemaphoreType.DMA,
        ],
        compiler_params=_SC_PARAMS,
    )


_msg1 = _mk_msg(False)
_msg2 = _mk_msg(True)


_RB = 1000
_PREC = jax.lax.Precision.HIGHEST


def _tc1_body(x_ref, w_ref, a_ref, h_ref, al_ref):
    h = lax.dot_general(x_ref[...], w_ref[...], (((1,), (0,)), ((), ())),
                        precision=_PREC, preferred_element_type=jnp.float32)
    h_ref[...] = h
    al_ref[...] = lax.dot_general(h, a_ref[...], (((1,), (0,)), ((), ())),
                                  precision=_PREC,
                                  preferred_element_type=jnp.float32)


_tc1 = pl.pallas_call(
    _tc1_body,
    grid=(N // _RB,),
    in_specs=[pl.BlockSpec((_RB, D), lambda i: (i, 0)),
              pl.BlockSpec((D, HC), lambda i: (0, 0)),
              pl.BlockSpec((HC, 2 * H), lambda i: (0, 0))],
    out_specs=[pl.BlockSpec((_RB, HC), lambda i: (i, 0)),
               pl.BlockSpec((_RB, 2 * H), lambda i: (i, 0))],
    out_shape=[jax.ShapeDtypeStruct((N, HC), jnp.float32),
               jax.ShapeDtypeStruct((N, 2 * H), jnp.float32)],
)


def _tc2_body(o0_ref, o1_ref, b_ref, w_ref, a_ref, h_ref, al_ref):
    g = o0_ref[...] + o1_ref[...] + b_ref[...]
    g = jnp.where(g > 0, g, jnp.exp(g) - 1.0)
    h = lax.dot_general(g, w_ref[...], (((1,), (0,)), ((), ())),
                        precision=_PREC, preferred_element_type=jnp.float32)
    h_ref[...] = h
    al_ref[...] = lax.dot_general(h, a_ref[...], (((1,), (0,)), ((), ())),
                                  precision=_PREC,
                                  preferred_element_type=jnp.float32)


_tc2 = pl.pallas_call(
    _tc2_body,
    grid=(N // _RB,),
    in_specs=[pl.BlockSpec((_RB, HC), lambda i: (i, 0)),
              pl.BlockSpec((_RB, HC), lambda i: (i, 0)),
              pl.BlockSpec((1, HC), lambda i: (0, 0)),
              pl.BlockSpec((HC, NCLS), lambda i: (0, 0)),
              pl.BlockSpec((NCLS, 2 * H), lambda i: (0, 0))],
    out_specs=[pl.BlockSpec((_RB, NCLS), lambda i: (i, 0)),
               pl.BlockSpec((_RB, 2 * H), lambda i: (i, 0))],
    out_shape=[jax.ShapeDtypeStruct((N, NCLS), jnp.float32),
               jax.ShapeDtypeStruct((N, 2 * H), jnp.float32)],
)


def _tc3_body(s0_ref, s1_ref, b_ref, o_ref):
    o_ref[...] = s0_ref[...] + s1_ref[...] + b_ref[...]


_tc3 = pl.pallas_call(
    _tc3_body,
    grid=(N // _RB,),
    in_specs=[pl.BlockSpec((_RB, NCLS), lambda i: (i, 0)),
              pl.BlockSpec((_RB, NCLS), lambda i: (i, 0)),
              pl.BlockSpec((1, NCLS), lambda i: (0, 0))],
    out_specs=pl.BlockSpec((_RB, NCLS), lambda i: (i, 0)),
    out_shape=jax.ShapeDtypeStruct((N, NCLS), jnp.float32),
)


def kernel(x, adjs, W1, a_src1, a_dst1, b1, W2, a_src2, a_dst2, b2):
    adjs = adjs.astype(jnp.int32)
    src = adjs[0]
    dst = adjs[1]

    mask8 = jnp.asarray(_MASK8)
    A1 = jnp.concatenate([mask8 * a_src1.reshape(HC, 1),
                          mask8 * a_dst1.reshape(HC, 1)], axis=1)
    z7 = jnp.zeros((NCLS, H - 1), jnp.float32)
    A2 = jnp.concatenate([a_src2.reshape(NCLS, 1), z7,
                          a_dst2.reshape(NCLS, 1), z7], axis=1)

    z8 = jnp.zeros((N, H), jnp.float32)
    z64 = jnp.zeros((N, NCLS), jnp.float32)

    h1, al1 = _tc1(x, W1, A1)
    as1 = al1[:, :H]
    ad1 = al1[:, H:]

    dens, ex1 = _att(src, dst, as1, ad1, z8)
    osum = _msg1(src, dst, ex1, dens[0], dens[1], h1, z64)

    h2, al2 = _tc2(osum[0], osum[1], b1.reshape(1, HC), W2, A2)
    as2 = al2[:, :H]
    ad2 = al2[:, H:]

    dens2, ex2 = _att(src, dst, as2, ad2, z8)
    osum2 = _msg2(src, dst, ex2, dens2[0], dens2[1], h2, z64)

    return _tc3(osum2[0], osum2[1], b2.reshape(1, NCLS))


# speedup vs baseline: 37.2230x; 1.2294x over previous
import numpy as np
import jax
import jax.numpy as jnp
from jax import lax
from jax.experimental import pallas as pl
from jax.experimental.pallas import tpu as pltpu
from jax.experimental.pallas import tpu_sc as plsc

N = 10000
E = 320000
D = 128
H = 8
CH = 8
HC = H * CH
NCLS = 64
SLOPE = 0.2
EPS = 1e-16

NC = 2
NS = 16
NW = NC * NS
EPW = E // NW
CB = 400
NCHUNK = EPW // CB
SUB = 5
SUBB = CB // SUB

_MASK8 = np.zeros((HC, H), np.float32)
for _h in range(H):
    _MASK8[_h * CH:(_h + 1) * CH, _h] = 1.0


def _iota16():
    return lax.iota(jnp.int32, 16)


def _splat(v):
    return jnp.full((16,), v, jnp.int32)


def _leaky_exp(e):
    return jnp.exp(jnp.where(e > 0, e, SLOPE * e))


_SC_PARAMS = pltpu.CompilerParams(needs_layout_passes=False,
                                  use_tc_tiling_on_sc=False)


def _att_body(src_hbm, dst_hbm, as_hbm, ad_hbm, z8_hbm,
              den_out, ex_out,
              sidx2, didx2, asr, adr, exc, accum, sem):
    cid = lax.axis_index("c")
    sid = lax.axis_index("s")
    wid = sid * NC + cid
    base = wid * EPW

    @pl.when(sid == 0)
    def _():
        pltpu.sync_copy(z8_hbm, accum)
    plsc.subcore_barrier()

    def chunk(c, _):
        off = base + c * CB
        roff = (base // SUBB) + c * SUB
        pltpu.sync_copy(src_hbm.at[pl.ds(roff, SUB), :], sidx2)
        pltpu.sync_copy(dst_hbm.at[pl.ds(roff, SUB), :], didx2)
        cps = []
        for j in range(SUB):
            isl = pl.ds(j * SUBB, SUBB)
            cps.append(pltpu.async_copy(as_hbm.at[sidx2.at[j]],
                                        asr.at[isl, :], sem))
            cps.append(pltpu.async_copy(ad_hbm.at[didx2.at[j]],
                                        adr.at[isl, :], sem))
        for cp in cps:
            cp.wait()

        lane = _iota16()
        half = lane >> 3
        mod8 = lane & 7

        def vec(k, _):
            row = _splat(2 * k) + half
            e = (plsc.load_gather(asr, [row, mod8])
                 + plsc.load_gather(adr, [row, mod8]))
            plsc.store_scatter(exc, [row, mod8], _leaky_exp(e))
            return 0
        lax.fori_loop(0, CB * H // 16, vec, 0)

        scps = []
        for j in range(SUB):
            scps.append(pltpu.async_copy(
                exc.at[pl.ds(j * SUBB, SUBB), :],
                accum.at[didx2.at[j]], sem, add=True))
        for cp in scps:
            cp.wait()
        pltpu.sync_copy(exc, ex_out.at[pl.ds(off, CB), :])
        return 0

    lax.fori_loop(0, NCHUNK, chunk, 0)
    plsc.subcore_barrier()

    @pl.when(sid == 0)
    def _():
        pltpu.sync_copy(accum, den_out.at[cid])


_att = pl.kernel(
    _att_body,
    out_type=(jax.ShapeDtypeStruct((NC, N, H), jnp.float32),
              jax.ShapeDtypeStruct((E, H), jnp.float32)),
    mesh=plsc.VectorSubcoreMesh(core_axis_name="c", subcore_axis_name="s"),
    scratch_types=[
        pltpu.VMEM((SUB, SUBB), jnp.int32),
        pltpu.VMEM((SUB, SUBB), jnp.int32),
        pltpu.VMEM((CB, H), jnp.float32),
        pltpu.VMEM((CB, H), jnp.float32),
        pltpu.VMEM((CB, H), jnp.float32),
        pltpu.VMEM_SHARED((N, H), jnp.float32),
        pltpu.SemaphoreType.DMA,
    ],
    compiler_params=_SC_PARAMS,
)


def _mk_msg(broadcast0):
    def body(src_hbm, dst_hbm, ex_hbm, d0_hbm, d1_hbm, h_hbm, z64_hbm,
             osum_out,
             sidx2, didx2, exc, d0r, d1r, alc, hr, msg, accum, sem):
        cid = lax.axis_index("c")
        sid = lax.axis_index("s")
        wid = sid * NC + cid
        base = wid * EPW

        lane = _iota16()
        half = lane >> 3
        mod8 = lane & 7

        @pl.when(sid == 0)
        def _():
            pltpu.sync_copy(z64_hbm, accum)
        plsc.subcore_barrier()

        def chunk(c, _):
            off = base + c * CB
            roff = (base // SUBB) + c * SUB
            pltpu.sync_copy(src_hbm.at[pl.ds(roff, SUB), :], sidx2)
            pltpu.sync_copy(dst_hbm.at[pl.ds(roff, SUB), :], didx2)
            cps = [pltpu.async_copy(ex_hbm.at[pl.ds(off, CB), :], exc, sem)]
            for j in range(SUB):
                isl = pl.ds(j * SUBB, SUBB)
                cps.append(pltpu.async_copy(d0_hbm.at[didx2.at[j]],
                                            d0r.at[isl, :], sem))
                cps.append(pltpu.async_copy(d1_hbm.at[didx2.at[j]],
                                            d1r.at[isl, :], sem))
                cps.append(pltpu.async_copy(h_hbm.at[sidx2.at[j]],
                                            hr.at[isl, :], sem))
            for cp in cps:
                cp.wait()

            def veca(k, _):
                row = _splat(2 * k) + half
                den = (plsc.load_gather(d0r, [row, mod8])
                       + plsc.load_gather(d1r, [row, mod8]))
                ex = plsc.load_gather(exc, [row, mod8])
                alc[pl.ds(16 * k, 16)] = ex / (den + EPS)
                return 0
            lax.fori_loop(0, CB * H // 16, veca, 0)

            def vecb(i, _):
                for t in range(2):
                    e = 2 * i + t
                    erow = _splat(e)
                    for j in range(4):
                        col = _splat(16 * j) + lane
                        if broadcast0:
                            aidx = _splat(H * e)
                        else:
                            aidx = _splat(H * e + 2 * j) + half
                        av = plsc.load_gather(alc, [aidx])
                        hv = plsc.load_gather(hr, [erow, col])
                        plsc.store_scatter(msg, [erow, col], hv * av)
                return 0
            lax.fori_loop(0, CB // 2, vecb, 0)

            scps = []
            for j in range(SUB):
                scps.append(pltpu.async_copy(
                    msg.at[pl.ds(j * SUBB, SUBB), :],
                    accum.at[didx2.at[j]], sem, add=True))
            for cp in scps:
                cp.wait()
            return 0

        lax.fori_loop(0, NCHUNK, chunk, 0)
        plsc.subcore_barrier()

        @pl.when(sid == 0)
        def _():
            pltpu.sync_copy(accum, osum_out.at[cid])

    return pl.kernel(
        body,
        out_type=jax.ShapeDtypeStruct((NC, N, HC), jnp.float32),
        mesh=plsc.VectorSubcoreMesh(core_axis_name="c", subcore_axis_name="s"),
        scratch_types=[
            pltpu.VMEM((SUB, SUBB), jnp.int32),
            pltpu.VMEM((SUB, SUBB), jnp.int32),
            pltpu.VMEM((CB, H), jnp.float32),
            pltpu.VMEM((CB, H), jnp.float32),
            pltpu.VMEM((CB, H), jnp.float32),
            pltpu.VMEM((CB * H,), jnp.float32),
            pltpu.VMEM((CB, HC), jnp.float32),
            pltpu.VMEM((CB, HC), jnp.float32),
            pltpu.VMEM_SHARED((N, HC), jnp.float32),
            pltpu.SemaphoreType.DMA,
        ],
        compiler_params=_SC_PARAMS,
    )


_msg1 = _mk_msg(False)
_msg2 = _mk_msg(True)


_RB = 1000
_PREC = jax.lax.Precision.HIGHEST


def _tc1_body(x_ref, w_ref, a_ref, h_ref, al_ref):
    h = lax.dot_general(x_ref[...], w_ref[...], (((1,), (0,)), ((), ())),
                        precision=_PREC, preferred_element_type=jnp.float32)
    h_ref[...] = h
    al_ref[...] = lax.dot_general(h, a_ref[...], (((1,), (0,)), ((), ())),
                                  precision=_PREC,
                                  preferred_element_type=jnp.float32)


_tc1 = pl.pallas_call(
    _tc1_body,
    grid=(N // _RB,),
    in_specs=[pl.BlockSpec((_RB, D), lambda i: (i, 0)),
              pl.BlockSpec((D, HC), lambda i: (0, 0)),
              pl.BlockSpec((HC, 2 * H), lambda i: (0, 0))],
    out_specs=[pl.BlockSpec((_RB, HC), lambda i: (i, 0)),
               pl.BlockSpec((_RB, 2 * H), lambda i: (i, 0))],
    out_shape=[jax.ShapeDtypeStruct((N, HC), jnp.float32),
               jax.ShapeDtypeStruct((N, 2 * H), jnp.float32)],
)


def _tc2_body(o0_ref, o1_ref, b_ref, w_ref, a_ref, h_ref, al_ref):
    g = o0_ref[...] + o1_ref[...] + b_ref[...]
    g = jnp.where(g > 0, g, jnp.exp(g) - 1.0)
    h = lax.dot_general(g, w_ref[...], (((1,), (0,)), ((), ())),
                        precision=_PREC, preferred_element_type=jnp.float32)
    h_ref[...] = h
    al_ref[...] = lax.dot_general(h, a_ref[...], (((1,), (0,)), ((), ())),
                                  precision=_PREC,
                                  preferred_element_type=jnp.float32)


_tc2 = pl.pallas_call(
    _tc2_body,
    grid=(N // _RB,),
    in_specs=[pl.BlockSpec((_RB, HC), lambda i: (i, 0)),
              pl.BlockSpec((_RB, HC), lambda i: (i, 0)),
              pl.BlockSpec((1, HC), lambda i: (0, 0)),
              pl.BlockSpec((HC, NCLS), lambda i: (0, 0)),
              pl.BlockSpec((NCLS, 2 * H), lambda i: (0, 0))],
    out_specs=[pl.BlockSpec((_RB, NCLS), lambda i: (i, 0)),
               pl.BlockSpec((_RB, 2 * H), lambda i: (i, 0))],
    out_shape=[jax.ShapeDtypeStruct((N, NCLS), jnp.float32),
               jax.ShapeDtypeStruct((N, 2 * H), jnp.float32)],
)


def _tc3_body(s0_ref, s1_ref, b_ref, o_ref):
    o_ref[...] = s0_ref[...] + s1_ref[...] + b_ref[...]


_tc3 = pl.pallas_call(
    _tc3_body,
    grid=(N // _RB,),
    in_specs=[pl.BlockSpec((_RB, NCLS), lambda i: (i, 0)),
              pl.BlockSpec((_RB, NCLS), lambda i: (i, 0)),
              pl.BlockSpec((1, NCLS), lambda i: (0, 0))],
    out_specs=pl.BlockSpec((_RB, NCLS), lambda i: (i, 0)),
    out_shape=jax.ShapeDtypeStruct((N, NCLS), jnp.float32),
)


def kernel(x, adjs, W1, a_src1, a_dst1, b1, W2, a_src2, a_dst2, b2):
    adjs = adjs.astype(jnp.int32)
    src = adjs[0].reshape(E // SUBB, SUBB)
    dst = adjs[1].reshape(E // SUBB, SUBB)

    mask8 = jnp.asarray(_MASK8)
    A1 = jnp.concatenate([mask8 * a_src1.reshape(HC, 1),
                          mask8 * a_dst1.reshape(HC, 1)], axis=1)
    z7 = jnp.zeros((NCLS, H - 1), jnp.float32)
    A2 = jnp.concatenate([a_src2.reshape(NCLS, 1), z7,
                          a_dst2.reshape(NCLS, 1), z7], axis=1)

    z8 = jnp.zeros((N, H), jnp.float32)
    z64 = jnp.zeros((N, NCLS), jnp.float32)

    h1, al1 = _tc1(x, W1, A1)
    as1 = al1[:, :H]
    ad1 = al1[:, H:]

    dens, ex1 = _att(src, dst, as1, ad1, z8)
    osum = _msg1(src, dst, ex1, dens[0], dens[1], h1, z64)

    h2, al2 = _tc2(osum[0], osum[1], b1.reshape(1, HC), W2, A2)
    as2 = al2[:, :H]
    ad2 = al2[:, H:]

    dens2, ex2 = _att(src, dst, as2, ad2, z8)
    osum2 = _msg2(src, dst, ex2, dens2[0], dens2[1], h2, z64)

    return _tc3(osum2[0], osum2[1], b2.reshape(1, NCLS))


# speedup vs baseline: 43.7020x; 1.1741x over previous
import numpy as np
import jax
import jax.numpy as jnp
from jax import lax
from jax.experimental import pallas as pl
from jax.experimental.pallas import tpu as pltpu
from jax.experimental.pallas import tpu_sc as plsc

N = 10000
E = 320000
D = 128
H = 8
CH = 8
HC = H * CH
NCLS = 64
SLOPE = 0.2
EPS = 1e-16

NC = 2
NS = 16
NW = NC * NS
EPW = E // NW
CB = 400
NCHUNK = EPW // CB
SUB = 5
SUBB = CB // SUB
NPAIR = (NCHUNK + 1) // 2

_MASK8 = np.zeros((HC, H), np.float32)
for _h in range(H):
    _MASK8[_h * CH:(_h + 1) * CH, _h] = 1.0


def _iota16():
    return lax.iota(jnp.int32, 16)


def _splat(v):
    return jnp.full((16,), v, jnp.int32)


def _leaky_exp(e):
    return jnp.exp(jnp.where(e > 0, e, SLOPE * e))


_SC_PARAMS = pltpu.CompilerParams(needs_layout_passes=False,
                                  use_tc_tiling_on_sc=False)


def _att_body(src_hbm, dst_hbm, as_hbm, ad_hbm, z8_hbm,
              den_out, ex_out,
              sA, dA, asrA, adrA, sB, dB, asrB, adrB,
              exc, accum, semA, semB):
    cid = lax.axis_index("c")
    sid = lax.axis_index("s")
    wid = sid * NC + cid
    base = wid * EPW
    rbase = base // SUBB

    bufs = [(sA, dA, asrA, adrA, semA), (sB, dB, asrB, adrB, semB)]

    lane = _iota16()
    half = lane >> 3
    mod8 = lane & 7

    @pl.when(sid == 0)
    def _():
        pltpu.sync_copy(z8_hbm, accum)
    plsc.subcore_barrier()

    def fire(c, bi):
        s2, d2, asr, adr, sem = bufs[bi]
        roff = rbase + c * SUB
        pltpu.sync_copy(src_hbm.at[pl.ds(roff, SUB), :], s2)
        pltpu.sync_copy(dst_hbm.at[pl.ds(roff, SUB), :], d2)
        for j in range(SUB):
            isl = pl.ds(j * SUBB, SUBB)
            pltpu.async_copy(as_hbm.at[s2.at[j]], asr.at[isl, :], sem)
            pltpu.async_copy(ad_hbm.at[d2.at[j]], adr.at[isl, :], sem)

    def wait(bi):
        s2, d2, asr, adr, sem = bufs[bi]
        for j in range(SUB):
            isl = pl.ds(j * SUBB, SUBB)
            pltpu.make_async_copy(as_hbm.at[s2.at[j]], asr.at[isl, :],
                                  sem).wait()
            pltpu.make_async_copy(ad_hbm.at[d2.at[j]], adr.at[isl, :],
                                  sem).wait()

    def compute(c, bi):
        s2, d2, asr, adr, sem = bufs[bi]

        def vec(k, _):
            row = _splat(2 * k) + half
            e = (plsc.load_gather(asr, [row, mod8])
                 + plsc.load_gather(adr, [row, mod8]))
            plsc.store_scatter(exc, [row, mod8], _leaky_exp(e))
            return 0
        lax.fori_loop(0, CB * H // 16, vec, 0)

        scps = []
        for j in range(SUB):
            scps.append(pltpu.async_copy(
                exc.at[pl.ds(j * SUBB, SUBB), :],
                accum.at[d2.at[j]], sem, add=True))
        for cp in scps:
            cp.wait()
        pltpu.sync_copy(exc, ex_out.at[pl.ds(base + c * CB, CB), :])

    fire(0, 0)

    def pair(g, _):
        c0 = 2 * g
        c1 = 2 * g + 1
        c2 = 2 * g + 2

        @pl.when(c1 < NCHUNK)
        def _():
            fire(c1, 1)
        wait(0)
        compute(c0, 0)

        @pl.when(c2 < NCHUNK)
        def _():
            fire(c2, 0)

        @pl.when(c1 < NCHUNK)
        def _():
            wait(1)
            compute(c1, 1)
        return 0

    lax.fori_loop(0, NPAIR, pair, 0)
    plsc.subcore_barrier()

    @pl.when(sid == 0)
    def _():
        pltpu.sync_copy(accum, den_out.at[cid])


_att = pl.kernel(
    _att_body,
    out_type=(jax.ShapeDtypeStruct((NC, N, H), jnp.float32),
              jax.ShapeDtypeStruct((E, H), jnp.float32)),
    mesh=plsc.VectorSubcoreMesh(core_axis_name="c", subcore_axis_name="s"),
    scratch_types=[
        pltpu.VMEM((SUB, SUBB), jnp.int32),
        pltpu.VMEM((SUB, SUBB), jnp.int32),
        pltpu.VMEM((CB, H), jnp.float32),
        pltpu.VMEM((CB, H), jnp.float32),
        pltpu.VMEM((SUB, SUBB), jnp.int32),
        pltpu.VMEM((SUB, SUBB), jnp.int32),
        pltpu.VMEM((CB, H), jnp.float32),
        pltpu.VMEM((CB, H), jnp.float32),
        pltpu.VMEM((CB, H), jnp.float32),
        pltpu.VMEM_SHARED((N, H), jnp.float32),
        pltpu.SemaphoreType.DMA,
        pltpu.SemaphoreType.DMA,
    ],
    compiler_params=_SC_PARAMS,
)


def _mk_msg(broadcast0):
    def body(src_hbm, dst_hbm, ex_hbm, den_hbm, h_hbm, z64_hbm,
             osum_out,
             sA, dA, excA, drA, hrA, sB, dB, excB, drB, hrB,
             accum, semA, semB):
        cid = lax.axis_index("c")
        sid = lax.axis_index("s")
        wid = sid * NC + cid
        base = wid * EPW
        rbase = base // SUBB

        bufs = [(sA, dA, excA, drA, hrA, semA),
                (sB, dB, excB, drB, hrB, semB)]

        lane = _iota16()
        half = lane >> 3
        mod8 = lane & 7

        @pl.when(sid == 0)
        def _():
            pltpu.sync_copy(z64_hbm, accum)
        plsc.subcore_barrier()

        def fire(c, bi):
            s2, d2, exc, dr, hr, sem = bufs[bi]
            roff = rbase + c * SUB
            pltpu.sync_copy(src_hbm.at[pl.ds(roff, SUB), :], s2)
            pltpu.sync_copy(dst_hbm.at[pl.ds(roff, SUB), :], d2)
            pltpu.async_copy(ex_hbm.at[pl.ds(base + c * CB, CB), :], exc, sem)
            for j in range(SUB):
                isl = pl.ds(j * SUBB, SUBB)
                pltpu.async_copy(den_hbm.at[d2.at[j]], dr.at[isl, :], sem)
                pltpu.async_copy(h_hbm.at[s2.at[j]], hr.at[isl, :], sem)

        def wait(bi):
            s2, d2, exc, dr, hr, sem = bufs[bi]
            pltpu.make_async_copy(ex_hbm.at[pl.ds(0, CB), :], exc, sem).wait()
            for j in range(SUB):
                isl = pl.ds(j * SUBB, SUBB)
                pltpu.make_async_copy(den_hbm.at[d2.at[j]], dr.at[isl, :],
                                      sem).wait()
                pltpu.make_async_copy(h_hbm.at[s2.at[j]], hr.at[isl, :],
                                      sem).wait()

        def compute(bi):
            s2, d2, exc, dr, hr, sem = bufs[bi]

            def veca(k, _):
                row = _splat(2 * k) + half
                den = plsc.load_gather(dr, [row, mod8])
                ex = plsc.load_gather(exc, [row, mod8])
                plsc.store_scatter(exc, [row, mod8], ex / (den + EPS))
                return 0
            lax.fori_loop(0, CB * H // 16, veca, 0)

            def vecb(i, _):
                for t in range(2):
                    e = 2 * i + t
                    erow = _splat(e)
                    for j in range(4):
                        col = _splat(16 * j) + lane
                        if broadcast0:
                            arow = _splat(e)
                            acol = _splat(0)
                        else:
                            arow = _splat(e)
                            acol = _splat(2 * j) + half
                        av = plsc.load_gather(exc, [arow, acol])
                        hv = plsc.load_gather(hr, [erow, col])
                        plsc.store_scatter(hr, [erow, col], hv * av)
                return 0
            lax.fori_loop(0, CB // 2, vecb, 0)

            scps = []
            for j in range(SUB):
                scps.append(pltpu.async_copy(
                    hr.at[pl.ds(j * SUBB, SUBB), :],
                    accum.at[d2.at[j]], sem, add=True))
            for cp in scps:
                cp.wait()

        fire(0, 0)

        def pair(g, _):
            c1 = 2 * g + 1
            c2 = 2 * g + 2

            @pl.when(c1 < NCHUNK)
            def _():
                fire(c1, 1)
            wait(0)
            compute(0)

            @pl.when(c2 < NCHUNK)
            def _():
                fire(c2, 0)

            @pl.when(c1 < NCHUNK)
            def _():
                wait(1)
                compute(1)
            return 0

        lax.fori_loop(0, NPAIR, pair, 0)
        plsc.subcore_barrier()

        @pl.when(sid == 0)
        def _():
            pltpu.sync_copy(accum, osum_out.at[cid])

    return pl.kernel(
        body,
        out_type=jax.ShapeDtypeStruct((NC, N, HC), jnp.float32),
        mesh=plsc.VectorSubcoreMesh(core_axis_name="c", subcore_axis_name="s"),
        scratch_types=[
            pltpu.VMEM((SUB, SUBB), jnp.int32),
            pltpu.VMEM((SUB, SUBB), jnp.int32),
            pltpu.VMEM((CB, H), jnp.float32),
            pltpu.VMEM((CB, H), jnp.float32),
            pltpu.VMEM((CB, HC), jnp.float32),
            pltpu.VMEM((SUB, SUBB), jnp.int32),
            pltpu.VMEM((SUB, SUBB), jnp.int32),
            pltpu.VMEM((CB, H), jnp.float32),
            pltpu.VMEM((CB, H), jnp.float32),
            pltpu.VMEM((CB, HC), jnp.float32),
            pltpu.VMEM_SHARED((N, HC), jnp.float32),
            pltpu.SemaphoreType.DMA,
            pltpu.SemaphoreType.DMA,
        ],
        compiler_params=_SC_PARAMS,
    )


_msg1 = _mk_msg(False)
_msg2 = _mk_msg(True)


_RB = 1000
_PREC = jax.lax.Precision.HIGHEST


def _tc1_body(x_ref, w_ref, a_ref, h_ref, al_ref):
    h = lax.dot_general(x_ref[...], w_ref[...], (((1,), (0,)), ((), ())),
                        precision=_PREC, preferred_element_type=jnp.float32)
    h_ref[...] = h
    al_ref[...] = lax.dot_general(h, a_ref[...], (((1,), (0,)), ((), ())),
                                  precision=_PREC,
                                  preferred_element_type=jnp.float32)


_tc1 = pl.pallas_call(
    _tc1_body,
    grid=(N // _RB,),
    in_specs=[pl.BlockSpec((_RB, D), lambda i: (i, 0)),
              pl.BlockSpec((D, HC), lambda i: (0, 0)),
              pl.BlockSpec((HC, 2 * H), lambda i: (0, 0))],
    out_specs=[pl.BlockSpec((_RB, HC), lambda i: (i, 0)),
               pl.BlockSpec((_RB, 2 * H), lambda i: (i, 0))],
    out_shape=[jax.ShapeDtypeStruct((N, HC), jnp.float32),
               jax.ShapeDtypeStruct((N, 2 * H), jnp.float32)],
)


def _tc2_body(o0_ref, o1_ref, b_ref, w_ref, a_ref, h_ref, al_ref):
    g = o0_ref[...] + o1_ref[...] + b_ref[...]
    g = jnp.where(g > 0, g, jnp.exp(g) - 1.0)
    h = lax.dot_general(g, w_ref[...], (((1,), (0,)), ((), ())),
                        precision=_PREC, preferred_element_type=jnp.float32)
    h_ref[...] = h
    al_ref[...] = lax.dot_general(h, a_ref[...], (((1,), (0,)), ((), ())),
                                  precision=_PREC,
                                  preferred_element_type=jnp.float32)


_tc2 = pl.pallas_call(
    _tc2_body,
    grid=(N // _RB,),
    in_specs=[pl.BlockSpec((_RB, HC), lambda i: (i, 0)),
              pl.BlockSpec((_RB, HC), lambda i: (i, 0)),
              pl.BlockSpec((1, HC), lambda i: (0, 0)),
              pl.BlockSpec((HC, NCLS), lambda i: (0, 0)),
              pl.BlockSpec((NCLS, 2 * H), lambda i: (0, 0))],
    out_specs=[pl.BlockSpec((_RB, NCLS), lambda i: (i, 0)),
               pl.BlockSpec((_RB, 2 * H), lambda i: (i, 0))],
    out_shape=[jax.ShapeDtypeStruct((N, NCLS), jnp.float32),
               jax.ShapeDtypeStruct((N, 2 * H), jnp.float32)],
)


def _tcd_body(d_ref, o_ref):
    o_ref[...] = d_ref[0] + d_ref[1]


_tcd = pl.pallas_call(
    _tcd_body,
    grid=(N // _RB,),
    in_specs=[pl.BlockSpec((NC, _RB, H), lambda i: (0, i, 0))],
    out_specs=pl.BlockSpec((_RB, H), lambda i: (i, 0)),
    out_shape=jax.ShapeDtypeStruct((N, H), jnp.float32),
)


def _tc3_body(s0_ref, s1_ref, b_ref, o_ref):
    o_ref[...] = s0_ref[...] + s1_ref[...] + b_ref[...]


_tc3 = pl.pallas_call(
    _tc3_body,
    grid=(N // _RB,),
    in_specs=[pl.BlockSpec((_RB, NCLS), lambda i: (i, 0)),
              pl.BlockSpec((_RB, NCLS), lambda i: (i, 0)),
              pl.BlockSpec((1, NCLS), lambda i: (0, 0))],
    out_specs=pl.BlockSpec((_RB, NCLS), lambda i: (i, 0)),
    out_shape=jax.ShapeDtypeStruct((N, NCLS), jnp.float32),
)


def kernel(x, adjs, W1, a_src1, a_dst1, b1, W2, a_src2, a_dst2, b2):
    adjs = adjs.astype(jnp.int32)
    src = adjs[0].reshape(E // SUBB, SUBB)
    dst = adjs[1].reshape(E // SUBB, SUBB)

    mask8 = jnp.asarray(_MASK8)
    A1 = jnp.concatenate([mask8 * a_src1.reshape(HC, 1),
                          mask8 * a_dst1.reshape(HC, 1)], axis=1)
    z7 = jnp.zeros((NCLS, H - 1), jnp.float32)
    A2 = jnp.concatenate([a_src2.reshape(NCLS, 1), z7,
                          a_dst2.reshape(NCLS, 1), z7], axis=1)

    z8 = jnp.zeros((N, H), jnp.float32)
    z64 = jnp.zeros((N, NCLS), jnp.float32)

    h1, al1 = _tc1(x, W1, A1)
    as1 = al1[:, :H]
    ad1 = al1[:, H:]

    dens, ex1 = _att(src, dst, as1, ad1, z8)
    osum = _msg1(src, dst, ex1, _tcd(dens), h1, z64)

    h2, al2 = _tc2(osum[0], osum[1], b1.reshape(1, HC), W2, A2)
    as2 = al2[:, :H]
    ad2 = al2[:, H:]

    dens2, ex2 = _att(src, dst, as2, ad2, z8)
    osum2 = _msg2(src, dst, ex2, _tcd(dens2), h2, z64)

    return _tc3(osum2[0], osum2[1], b2.reshape(1, NCLS))


# speedup vs baseline: 45.9160x; 1.0507x over previous
import numpy as np
import jax
import jax.numpy as jnp
from jax import lax
from jax.experimental import pallas as pl
from jax.experimental.pallas import tpu as pltpu
from jax.experimental.pallas import tpu_sc as plsc

N = 10000
E = 320000
D = 128
H = 8
CH = 8
HC = H * CH
NCLS = 64
SLOPE = 0.2
EPS = 1e-16

NC = 2
NS = 16
NW = NC * NS
EPW = E // NW
CB = 400
NCHUNK = EPW // CB
SUB = 10
SUBB = CB // SUB
HSUB = SUB // 2
HCB = CB // 2
NPAIR = (NCHUNK + 1) // 2

_MASK8 = np.zeros((HC, H), np.float32)
for _h in range(H):
    _MASK8[_h * CH:(_h + 1) * CH, _h] = 1.0


def _iota16():
    return lax.iota(jnp.int32, 16)


def _splat(v):
    return jnp.full((16,), v, jnp.int32)


def _leaky_exp(e):
    return jnp.exp(jnp.where(e > 0, e, SLOPE * e))


_SC_PARAMS = pltpu.CompilerParams(needs_layout_passes=False,
                                  use_tc_tiling_on_sc=False)


def _att_body(src_hbm, dst_hbm, as_hbm, ad_hbm, z8_hbm,
              den_out, ex_out,
              sA, dA, asrA, adrA, sB, dB, asrB, adrB,
              exc, accum, semA, semB):
    cid = lax.axis_index("c")
    sid = lax.axis_index("s")
    wid = sid * NC + cid
    base = wid * EPW
    rbase = base // SUBB

    bufs = [(sA, dA, asrA, adrA, semA), (sB, dB, asrB, adrB, semB)]

    lane = _iota16()
    half = lane >> 3
    mod8 = lane & 7

    @pl.when(sid == 0)
    def _():
        pltpu.sync_copy(z8_hbm, accum)
    plsc.subcore_barrier()

    def fire(c, bi):
        s2, d2, asr, adr, sem = bufs[bi]
        roff = rbase + c * SUB
        pltpu.sync_copy(src_hbm.at[pl.ds(roff, SUB), :], s2)
        pltpu.sync_copy(dst_hbm.at[pl.ds(roff, SUB), :], d2)
        for j in range(SUB):
            isl = pl.ds(j * SUBB, SUBB)
            pltpu.async_copy(as_hbm.at[s2.at[j]], asr.at[isl, :], sem)
            pltpu.async_copy(ad_hbm.at[d2.at[j]], adr.at[isl, :], sem)

    def wait(bi):
        s2, d2, asr, adr, sem = bufs[bi]
        for j in range(SUB):
            isl = pl.ds(j * SUBB, SUBB)
            pltpu.make_async_copy(as_hbm.at[s2.at[j]], asr.at[isl, :],
                                  sem).wait()
            pltpu.make_async_copy(ad_hbm.at[d2.at[j]], adr.at[isl, :],
                                  sem).wait()

    def compute(c, bi):
        s2, d2, asr, adr, sem = bufs[bi]

        def vec(k, _):
            row = _splat(2 * k) + half
            e = (plsc.load_gather(asr, [row, mod8])
                 + plsc.load_gather(adr, [row, mod8]))
            plsc.store_scatter(exc, [row, mod8], _leaky_exp(e))
            return 0
        lax.fori_loop(0, CB * H // 16, vec, 0)

        scps = []
        for j in range(SUB):
            scps.append(pltpu.async_copy(
                exc.at[pl.ds(j * SUBB, SUBB), :],
                accum.at[d2.at[j]], sem, add=True))
        for cp in scps:
            cp.wait()
        pltpu.sync_copy(exc, ex_out.at[pl.ds(base + c * CB, CB), :])

    fire(0, 0)

    def pair(g, _):
        c0 = 2 * g
        c1 = 2 * g + 1
        c2 = 2 * g + 2

        @pl.when(c1 < NCHUNK)
        def _():
            fire(c1, 1)
        wait(0)
        compute(c0, 0)

        @pl.when(c2 < NCHUNK)
        def _():
            fire(c2, 0)

        @pl.when(c1 < NCHUNK)
        def _():
            wait(1)
            compute(c1, 1)
        return 0

    lax.fori_loop(0, NPAIR, pair, 0)
    plsc.subcore_barrier()

    @pl.when(sid == 0)
    def _():
        pltpu.sync_copy(accum, den_out.at[cid])


_att = pl.kernel(
    _att_body,
    out_type=(jax.ShapeDtypeStruct((NC, N, H), jnp.float32),
              jax.ShapeDtypeStruct((E, H), jnp.float32)),
    mesh=plsc.VectorSubcoreMesh(core_axis_name="c", subcore_axis_name="s"),
    scratch_types=[
        pltpu.VMEM((SUB, SUBB), jnp.int32),
        pltpu.VMEM((SUB, SUBB), jnp.int32),
        pltpu.VMEM((CB, H), jnp.float32),
        pltpu.VMEM((CB, H), jnp.float32),
        pltpu.VMEM((SUB, SUBB), jnp.int32),
        pltpu.VMEM((SUB, SUBB), jnp.int32),
        pltpu.VMEM((CB, H), jnp.float32),
        pltpu.VMEM((CB, H), jnp.float32),
        pltpu.VMEM((CB, H), jnp.float32),
        pltpu.VMEM_SHARED((N, H), jnp.float32),
        pltpu.SemaphoreType.DMA,
        pltpu.SemaphoreType.DMA,
    ],
    compiler_params=_SC_PARAMS,
)


def _mk_msg(broadcast0):
    def body(src_hbm, dst_hbm, ex_hbm, den_hbm, h_hbm, z64_hbm,
             osum_out,
             sA, dA, excA, drA, hrA, sB, dB, excB, drB, hrB,
             msg, accum, semA, semB):
        cid = lax.axis_index("c")
        sid = lax.axis_index("s")
        wid = sid * NC + cid
        base = wid * EPW
        rbase = base // SUBB

        bufs = [(sA, dA, excA, drA, hrA, semA),
                (sB, dB, excB, drB, hrB, semB)]

        lane = _iota16()
        half = lane >> 3
        mod8 = lane & 7

        @pl.when(sid == 0)
        def _():
            pltpu.sync_copy(z64_hbm, accum)
        plsc.subcore_barrier()

        def fire(c, bi):
            s2, d2, exc, dr, hr, sem = bufs[bi]
            roff = rbase + c * SUB
            pltpu.sync_copy(src_hbm.at[pl.ds(roff, SUB), :], s2)
            pltpu.sync_copy(dst_hbm.at[pl.ds(roff, SUB), :], d2)
            pltpu.async_copy(ex_hbm.at[pl.ds(base + c * CB, CB), :], exc, sem)
            for j in range(SUB):
                isl = pl.ds(j * SUBB, SUBB)
                pltpu.async_copy(den_hbm.at[d2.at[j]], dr.at[isl, :], sem)
                pltpu.async_copy(h_hbm.at[s2.at[j]], hr.at[isl, :], sem)

        def wait(bi):
            s2, d2, exc, dr, hr, sem = bufs[bi]
            pltpu.make_async_copy(ex_hbm.at[pl.ds(0, CB), :], exc, sem).wait()
            for j in range(SUB):
                isl = pl.ds(j * SUBB, SUBB)
                pltpu.make_async_copy(den_hbm.at[d2.at[j]], dr.at[isl, :],
                                      sem).wait()
                pltpu.make_async_copy(h_hbm.at[s2.at[j]], hr.at[isl, :],
                                      sem).wait()

        acol = [_splat(2 * j) + half for j in range(4)]
        zcol = _splat(0)

        def compute(bi):
            s2, d2, exc, dr, hr, sem = bufs[bi]

            def veca(k, _):
                row = _splat(2 * k) + half
                den = plsc.load_gather(dr, [row, mod8])
                ex = plsc.load_gather(exc, [row, mod8])
                plsc.store_scatter(exc, [row, mod8], ex / (den + EPS))
                return 0
            lax.fori_loop(0, CB * H // 16, veca, 0)

            for hb in range(2):
                eb = hb * HCB

                def vecb(i, _):
                    for t in range(2):
                        r = 2 * i + t
                        e = eb + r
                        if broadcast0:
                            av = plsc.load_gather(exc, [_splat(e), zcol])
                            for j in range(4):
                                sl = pl.ds(16 * j, 16)
                                msg[r, sl] = hr[e, sl] * av
                        else:
                            arow = _splat(e)
                            for j in range(4):
                                sl = pl.ds(16 * j, 16)
                                av = plsc.load_gather(exc, [arow, acol[j]])
                                msg[r, sl] = hr[e, sl] * av
                    return 0
                lax.fori_loop(0, HCB // 2, vecb, 0)

                scps = []
                for j in range(HSUB):
                    scps.append(pltpu.async_copy(
                        msg.at[pl.ds(j * SUBB, SUBB), :],
                        accum.at[d2.at[hb * HSUB + j]], sem, add=True))
                for cp in scps:
                    cp.wait()

        fire(0, 0)

        def pair(g, _):
            c1 = 2 * g + 1
            c2 = 2 * g + 2

            @pl.when(c1 < NCHUNK)
            def _():
                fire(c1, 1)
            wait(0)
            compute(0)

            @pl.when(c2 < NCHUNK)
            def _():
                fire(c2, 0)

            @pl.when(c1 < NCHUNK)
            def _():
                wait(1)
                compute(1)
            return 0

        lax.fori_loop(0, NPAIR, pair, 0)
        plsc.subcore_barrier()

        @pl.when(sid == 0)
        def _():
            pltpu.sync_copy(accum, osum_out.at[cid])

    return pl.kernel(
        body,
        out_type=jax.ShapeDtypeStruct((NC, N, HC), jnp.float32),
        mesh=plsc.VectorSubcoreMesh(core_axis_name="c", subcore_axis_name="s"),
        scratch_types=[
            pltpu.VMEM((SUB, SUBB), jnp.int32),
            pltpu.VMEM((SUB, SUBB), jnp.int32),
            pltpu.VMEM((CB, H), jnp.float32),
            pltpu.VMEM((CB, H), jnp.float32),
            pltpu.VMEM((CB, HC), jnp.float32),
            pltpu.VMEM((SUB, SUBB), jnp.int32),
            pltpu.VMEM((SUB, SUBB), jnp.int32),
            pltpu.VMEM((CB, H), jnp.float32),
            pltpu.VMEM((CB, H), jnp.float32),
            pltpu.VMEM((CB, HC), jnp.float32),
            pltpu.VMEM((HCB, HC), jnp.float32),
            pltpu.VMEM_SHARED((N, HC), jnp.float32),
            pltpu.SemaphoreType.DMA,
            pltpu.SemaphoreType.DMA,
        ],
        compiler_params=_SC_PARAMS,
    )


_msg1 = _mk_msg(False)
_msg2 = _mk_msg(True)


_RB = 1000
_PREC = jax.lax.Precision.HIGHEST


def _tc1_body(x_ref, w_ref, a_ref, h_ref, al_ref):
    h = lax.dot_general(x_ref[...], w_ref[...], (((1,), (0,)), ((), ())),
                        precision=_PREC, preferred_element_type=jnp.float32)
    h_ref[...] = h
    al_ref[...] = lax.dot_general(h, a_ref[...], (((1,), (0,)), ((), ())),
                                  precision=_PREC,
                                  preferred_element_type=jnp.float32)


_tc1 = pl.pallas_call(
    _tc1_body,
    grid=(N // _RB,),
    in_specs=[pl.BlockSpec((_RB, D), lambda i: (i, 0)),
              pl.BlockSpec((D, HC), lambda i: (0, 0)),
              pl.BlockSpec((HC, 2 * H), lambda i: (0, 0))],
    out_specs=[pl.BlockSpec((_RB, HC), lambda i: (i, 0)),
               pl.BlockSpec((_RB, 2 * H), lambda i: (i, 0))],
    out_shape=[jax.ShapeDtypeStruct((N, HC), jnp.float32),
               jax.ShapeDtypeStruct((N, 2 * H), jnp.float32)],
)


def _tc2_body(o0_ref, o1_ref, b_ref, w_ref, a_ref, h_ref, al_ref):
    g = o0_ref[...] + o1_ref[...] + b_ref[...]
    g = jnp.where(g > 0, g, jnp.exp(g) - 1.0)
    h = lax.dot_general(g, w_ref[...], (((1,), (0,)), ((), ())),
                        precision=_PREC, preferred_element_type=jnp.float32)
    h_ref[...] = h
    al_ref[...] = lax.dot_general(h, a_ref[...], (((1,), (0,)), ((), ())),
                                  precision=_PREC,
                                  preferred_element_type=jnp.float32)


_tc2 = pl.pallas_call(
    _tc2_body,
    grid=(N // _RB,),
    in_specs=[pl.BlockSpec((_RB, HC), lambda i: (i, 0)),
              pl.BlockSpec((_RB, HC), lambda i: (i, 0)),
              pl.BlockSpec((1, HC), lambda i: (0, 0)),
              pl.BlockSpec((HC, NCLS), lambda i: (0, 0)),
              pl.BlockSpec((NCLS, 2 * H), lambda i: (0, 0))],
    out_specs=[pl.BlockSpec((_RB, NCLS), lambda i: (i, 0)),
               pl.BlockSpec((_RB, 2 * H), lambda i: (i, 0))],
    out_shape=[jax.ShapeDtypeStruct((N, NCLS), jnp.float32),
               jax.ShapeDtypeStruct((N, 2 * H), jnp.float32)],
)


def _tcd_body(d_ref, o_ref):
    o_ref[...] = d_ref[0] + d_ref[1]


_tcd = pl.pallas_call(
    _tcd_body,
    grid=(N // _RB,),
    in_specs=[pl.BlockSpec((NC, _RB, H), lambda i: (0, i, 0))],
    out_specs=pl.BlockSpec((_RB, H), lambda i: (i, 0)),
    out_shape=jax.ShapeDtypeStruct((N, H), jnp.float32),
)


def _tc3_body(s0_ref, s1_ref, b_ref, o_ref):
    o_ref[...] = s0_ref[...] + s1_ref[...] + b_ref[...]


_tc3 = pl.pallas_call(
    _tc3_body,
    grid=(N // _RB,),
    in_specs=[pl.BlockSpec((_RB, NCLS), lambda i: (i, 0)),
              pl.BlockSpec((_RB, NCLS), lambda i: (i, 0)),
              pl.BlockSpec((1, NCLS), lambda i: (0, 0))],
    out_specs=pl.BlockSpec((_RB, NCLS), lambda i: (i, 0)),
    out_shape=jax.ShapeDtypeStruct((N, NCLS), jnp.float32),
)


def kernel(x, adjs, W1, a_src1, a_dst1, b1, W2, a_src2, a_dst2, b2):
    adjs = adjs.astype(jnp.int32)
    src = adjs[0].reshape(E // SUBB, SUBB)
    dst = adjs[1].reshape(E // SUBB, SUBB)

    mask8 = jnp.asarray(_MASK8)
    A1 = jnp.concatenate([mask8 * a_src1.reshape(HC, 1),
                          mask8 * a_dst1.reshape(HC, 1)], axis=1)
    z7 = jnp.zeros((NCLS, H - 1), jnp.float32)
    A2 = jnp.concatenate([a_src2.reshape(NCLS, 1), z7,
                          a_dst2.reshape(NCLS, 1), z7], axis=1)

    z8 = jnp.zeros((N, H), jnp.float32)
    z64 = jnp.zeros((N, NCLS), jnp.float32)

    h1, al1 = _tc1(x, W1, A1)
    as1 = al1[:, :H]
    ad1 = al1[:, H:]

    dens, ex1 = _att(src, dst, as1, ad1, z8)
    osum = _msg1(src, dst, ex1, _tcd(dens), h1, z64)

    h2, al2 = _tc2(osum[0], osum[1], b1.reshape(1, HC), W2, A2)
    as2 = al2[:, :H]
    ad2 = al2[:, H:]

    dens2, ex2 = _att(src, dst, as2, ad2, z8)
    osum2 = _msg2(src, dst, ex2, _tcd(dens2), h2, z64)

    return _tc3(osum2[0], osum2[1], b2.reshape(1, NCLS))


# speedup vs baseline: 46.7727x; 1.0187x over previous
import numpy as np
import jax
import jax.numpy as jnp
from jax import lax
from jax.experimental import pallas as pl
from jax.experimental.pallas import tpu as pltpu
from jax.experimental.pallas import tpu_sc as plsc

N = 10000
E = 320000
D = 128
H = 8
CH = 8
HC = H * CH
NCLS = 64
SLOPE = 0.2
EPS = 1e-16

NC = 2
NS = 16
NW = NC * NS
EPW = E // NW
CB = 400
NCHUNK = EPW // CB
SUB = 10
SUBB = CB // SUB
HSUB = SUB // 2
HCB = CB // 2

CBA = 1000
NCHUNKA = EPW // CBA
SUBA = 10
SUBBA = CBA // SUBA
NPAIRA = (NCHUNKA + 1) // 2
NPAIR = (NCHUNK + 1) // 2

_MASK8 = np.zeros((HC, H), np.float32)
for _h in range(H):
    _MASK8[_h * CH:(_h + 1) * CH, _h] = 1.0


def _iota16():
    return lax.iota(jnp.int32, 16)


def _splat(v):
    return jnp.full((16,), v, jnp.int32)


def _leaky_exp(e):
    return jnp.exp(jnp.where(e > 0, e, SLOPE * e))


_SC_PARAMS = pltpu.CompilerParams(needs_layout_passes=False,
                                  use_tc_tiling_on_sc=False)


def _att_body(src_hbm, dst_hbm, as_hbm, ad_hbm, z8_hbm,
              den_out, ex_out,
              sA, dA, asrA, adrA, sB, dB, asrB, adrB,
              exc, accum, semA, semB):
    cid = lax.axis_index("c")
    sid = lax.axis_index("s")
    wid = sid * NC + cid
    base = wid * EPW
    rbase = base // SUBBA

    bufs = [(sA, dA, asrA, adrA, semA), (sB, dB, asrB, adrB, semB)]

    lane = _iota16()
    half = lane >> 3
    mod8 = lane & 7

    @pl.when(sid == 0)
    def _():
        pltpu.sync_copy(z8_hbm, accum)
    plsc.subcore_barrier()

    def fire(c, bi):
        s2, d2, asr, adr, sem = bufs[bi]
        roff = rbase + c * SUBA
        pltpu.sync_copy(src_hbm.at[pl.ds(roff, SUBA), :], s2)
        pltpu.sync_copy(dst_hbm.at[pl.ds(roff, SUBA), :], d2)
        for j in range(SUBA):
            isl = pl.ds(j * SUBBA, SUBBA)
            pltpu.async_copy(as_hbm.at[s2.at[j]], asr.at[isl, :], sem)
            pltpu.async_copy(ad_hbm.at[d2.at[j]], adr.at[isl, :], sem)

    def wait(bi):
        s2, d2, asr, adr, sem = bufs[bi]
        for j in range(SUBA):
            isl = pl.ds(j * SUBBA, SUBBA)
            pltpu.make_async_copy(as_hbm.at[s2.at[j]], asr.at[isl, :],
                                  sem).wait()
            pltpu.make_async_copy(ad_hbm.at[d2.at[j]], adr.at[isl, :],
                                  sem).wait()

    def compute(c, bi):
        s2, d2, asr, adr, sem = bufs[bi]

        def vec(k, _):
            row = _splat(2 * k) + half
            e = (plsc.load_gather(asr, [row, mod8])
                 + plsc.load_gather(adr, [row, mod8]))
            plsc.store_scatter(exc, [row, mod8], _leaky_exp(e))
            return 0
        lax.fori_loop(0, CBA * H // 16, vec, 0)

        scps = []
        for j in range(SUBA):
            scps.append(pltpu.async_copy(
                exc.at[pl.ds(j * SUBBA, SUBBA), :],
                accum.at[d2.at[j]], sem, add=True))
        for cp in scps:
            cp.wait()
        pltpu.sync_copy(exc, ex_out.at[pl.ds(base + c * CBA, CBA), :])

    fire(0, 0)

    def pair(g, _):
        c0 = 2 * g
        c1 = 2 * g + 1
        c2 = 2 * g + 2

        @pl.when(c1 < NCHUNKA)
        def _():
            fire(c1, 1)
        wait(0)
        compute(c0, 0)

        @pl.when(c2 < NCHUNKA)
        def _():
            fire(c2, 0)

        @pl.when(c1 < NCHUNKA)
        def _():
            wait(1)
            compute(c1, 1)
        return 0

    lax.fori_loop(0, NPAIRA, pair, 0)
    plsc.subcore_barrier()

    @pl.when(sid == 0)
    def _():
        pltpu.sync_copy(accum, den_out.at[cid])


_att = pl.kernel(
    _att_body,
    out_type=(jax.ShapeDtypeStruct((NC, N, H), jnp.float32),
              jax.ShapeDtypeStruct((E, H), jnp.float32)),
    mesh=plsc.VectorSubcoreMesh(core_axis_name="c", subcore_axis_name="s"),
    scratch_types=[
        pltpu.VMEM((SUBA, SUBBA), jnp.int32),
        pltpu.VMEM((SUBA, SUBBA), jnp.int32),
        pltpu.VMEM((CBA, H), jnp.float32),
        pltpu.VMEM((CBA, H), jnp.float32),
        pltpu.VMEM((SUBA, SUBBA), jnp.int32),
        pltpu.VMEM((SUBA, SUBBA), jnp.int32),
        pltpu.VMEM((CBA, H), jnp.float32),
        pltpu.VMEM((CBA, H), jnp.float32),
        pltpu.VMEM((CBA, H), jnp.float32),
        pltpu.VMEM_SHARED((N, H), jnp.float32),
        pltpu.SemaphoreType.DMA,
        pltpu.SemaphoreType.DMA,
    ],
    compiler_params=_SC_PARAMS,
)


def _mk_msg(broadcast0):
    def body(src_hbm, dst_hbm, ex_hbm, den_hbm, h_hbm, z64_hbm,
             osum_out,
             sA, dA, excA, drA, hrA, sB, dB, excB, drB, hrB,
             msg, accum, semA, semB):
        cid = lax.axis_index("c")
        sid = lax.axis_index("s")
        wid = sid * NC + cid
        base = wid * EPW
        rbase = base // SUBB

        bufs = [(sA, dA, excA, drA, hrA, semA),
                (sB, dB, excB, drB, hrB, semB)]

        lane = _iota16()
        half = lane >> 3
        mod8 = lane & 7

        @pl.when(sid == 0)
        def _():
            pltpu.sync_copy(z64_hbm, accum)
        plsc.subcore_barrier()

        def fire(c, bi):
            s2, d2, exc, dr, hr, sem = bufs[bi]
            roff = rbase + c * SUB
            pltpu.sync_copy(src_hbm.at[pl.ds(roff, SUB), :], s2)
            pltpu.sync_copy(dst_hbm.at[pl.ds(roff, SUB), :], d2)
            pltpu.async_copy(ex_hbm.at[pl.ds(base + c * CB, CB), :], exc, sem)
            for j in range(SUB):
                isl = pl.ds(j * SUBB, SUBB)
                pltpu.async_copy(den_hbm.at[d2.at[j]], dr.at[isl, :], sem)
                pltpu.async_copy(h_hbm.at[s2.at[j]], hr.at[isl, :], sem)

        def wait(bi):
            s2, d2, exc, dr, hr, sem = bufs[bi]
            pltpu.make_async_copy(ex_hbm.at[pl.ds(0, CB), :], exc, sem).wait()
            for j in range(SUB):
                isl = pl.ds(j * SUBB, SUBB)
                pltpu.make_async_copy(den_hbm.at[d2.at[j]], dr.at[isl, :],
                                      sem).wait()
                pltpu.make_async_copy(h_hbm.at[s2.at[j]], hr.at[isl, :],
                                      sem).wait()

        acol = [_splat(2 * j) + half for j in range(4)]
        zcol = _splat(0)

        def compute(bi):
            s2, d2, exc, dr, hr, sem = bufs[bi]

            def veca(k, _):
                row = _splat(2 * k) + half
                den = plsc.load_gather(dr, [row, mod8])
                ex = plsc.load_gather(exc, [row, mod8])
                plsc.store_scatter(exc, [row, mod8], ex / (den + EPS))
                return 0
            lax.fori_loop(0, CB * H // 16, veca, 0)

            for hb in range(2):
                eb = hb * HCB

                def vecb(i, _):
                    for t in range(2):
                        r = 2 * i + t
                        e = eb + r
                        if broadcast0:
                            av = plsc.load_gather(exc, [_splat(e), zcol])
                            for j in range(4):
                                sl = pl.ds(16 * j, 16)
                                msg[r, sl] = hr[e, sl] * av
                        else:
                            arow = _splat(e)
                            for j in range(4):
                                sl = pl.ds(16 * j, 16)
                                av = plsc.load_gather(exc, [arow, acol[j]])
                                msg[r, sl] = hr[e, sl] * av
                    return 0
                lax.fori_loop(0, HCB // 2, vecb, 0)

                scps = []
                for j in range(HSUB):
                    scps.append(pltpu.async_copy(
                        msg.at[pl.ds(j * SUBB, SUBB), :],
                        accum.at[d2.at[hb * HSUB + j]], sem, add=True))
                for cp in scps:
                    cp.wait()

        fire(0, 0)

        def pair(g, _):
            c1 = 2 * g + 1
            c2 = 2 * g + 2

            @pl.when(c1 < NCHUNK)
            def _():
                fire(c1, 1)
            wait(0)
            compute(0)

            @pl.when(c2 < NCHUNK)
            def _():
                fire(c2, 0)

            @pl.when(c1 < NCHUNK)
            def _():
                wait(1)
                compute(1)
            return 0

        lax.fori_loop(0, NPAIR, pair, 0)
        plsc.subcore_barrier()

        @pl.when(sid == 0)
        def _():
            pltpu.sync_copy(accum, osum_out.at[cid])

    return pl.kernel(
        body,
        out_type=jax.ShapeDtypeStruct((NC, N, HC), jnp.float32),
        mesh=plsc.VectorSubcoreMesh(core_axis_name="c", subcore_axis_name="s"),
        scratch_types=[
            pltpu.VMEM((SUB, SUBB), jnp.int32),
            pltpu.VMEM((SUB, SUBB), jnp.int32),
            pltpu.VMEM((CB, H), jnp.float32),
            pltpu.VMEM((CB, H), jnp.float32),
            pltpu.VMEM((CB, HC), jnp.float32),
            pltpu.VMEM((SUB, SUBB), jnp.int32),
            pltpu.VMEM((SUB, SUBB), jnp.int32),
            pltpu.VMEM((CB, H), jnp.float32),
            pltpu.VMEM((CB, H), jnp.float32),
            pltpu.VMEM((CB, HC), jnp.float32),
            pltpu.VMEM((HCB, HC), jnp.float32),
            pltpu.VMEM_SHARED((N, HC), jnp.float32),
            pltpu.SemaphoreType.DMA,
            pltpu.SemaphoreType.DMA,
        ],
        compiler_params=_SC_PARAMS,
    )


_msg1 = _mk_msg(False)
_msg2 = _mk_msg(True)


_RB = 1000
_PREC = jax.lax.Precision.HIGHEST


def _tc1_body(x_ref, w_ref, a_ref, h_ref, al_ref):
    h = lax.dot_general(x_ref[...], w_ref[...], (((1,), (0,)), ((), ())),
                        precision=_PREC, preferred_element_type=jnp.float32)
    h_ref[...] = h
    al_ref[...] = lax.dot_general(h, a_ref[...], (((1,), (0,)), ((), ())),
                                  precision=_PREC,
                                  preferred_element_type=jnp.float32)


_tc1 = pl.pallas_call(
    _tc1_body,
    grid=(N // _RB,),
    in_specs=[pl.BlockSpec((_RB, D), lambda i: (i, 0)),
              pl.BlockSpec((D, HC), lambda i: (0, 0)),
              pl.BlockSpec((HC, 2 * H), lambda i: (0, 0))],
    out_specs=[pl.BlockSpec((_RB, HC), lambda i: (i, 0)),
               pl.BlockSpec((_RB, 2 * H), lambda i: (i, 0))],
    out_shape=[jax.ShapeDtypeStruct((N, HC), jnp.float32),
               jax.ShapeDtypeStruct((N, 2 * H), jnp.float32)],
)


def _tc2_body(o0_ref, o1_ref, b_ref, w_ref, a_ref, h_ref, al_ref):
    g = o0_ref[...] + o1_ref[...] + b_ref[...]
    g = jnp.where(g > 0, g, jnp.exp(g) - 1.0)
    h = lax.dot_general(g, w_ref[...], (((1,), (0,)), ((), ())),
                        precision=_PREC, preferred_element_type=jnp.float32)
    h_ref[...] = h
    al_ref[...] = lax.dot_general(h, a_ref[...], (((1,), (0,)), ((), ())),
                                  precision=_PREC,
                                  preferred_element_type=jnp.float32)


_tc2 = pl.pallas_call(
    _tc2_body,
    grid=(N // _RB,),
    in_specs=[pl.BlockSpec((_RB, HC), lambda i: (i, 0)),
              pl.BlockSpec((_RB, HC), lambda i: (i, 0)),
              pl.BlockSpec((1, HC), lambda i: (0, 0)),
              pl.BlockSpec((HC, NCLS), lambda i: (0, 0)),
              pl.BlockSpec((NCLS, 2 * H), lambda i: (0, 0))],
    out_specs=[pl.BlockSpec((_RB, NCLS), lambda i: (i, 0)),
               pl.BlockSpec((_RB, 2 * H), lambda i: (i, 0))],
    out_shape=[jax.ShapeDtypeStruct((N, NCLS), jnp.float32),
               jax.ShapeDtypeStruct((N, 2 * H), jnp.float32)],
)


def _tcd_body(d_ref, o_ref):
    o_ref[...] = d_ref[0] + d_ref[1]


_tcd = pl.pallas_call(
    _tcd_body,
    grid=(N // _RB,),
    in_specs=[pl.BlockSpec((NC, _RB, H), lambda i: (0, i, 0))],
    out_specs=pl.BlockSpec((_RB, H), lambda i: (i, 0)),
    out_shape=jax.ShapeDtypeStruct((N, H), jnp.float32),
)


def _tc3_body(s0_ref, s1_ref, b_ref, o_ref):
    o_ref[...] = s0_ref[...] + s1_ref[...] + b_ref[...]


_tc3 = pl.pallas_call(
    _tc3_body,
    grid=(N // _RB,),
    in_specs=[pl.BlockSpec((_RB, NCLS), lambda i: (i, 0)),
              pl.BlockSpec((_RB, NCLS), lambda i: (i, 0)),
              pl.BlockSpec((1, NCLS), lambda i: (0, 0))],
    out_specs=pl.BlockSpec((_RB, NCLS), lambda i: (i, 0)),
    out_shape=jax.ShapeDtypeStruct((N, NCLS), jnp.float32),
)


def kernel(x, adjs, W1, a_src1, a_dst1, b1, W2, a_src2, a_dst2, b2):
    adjs = adjs.astype(jnp.int32)
    src = adjs[0].reshape(E // SUBB, SUBB)
    dst = adjs[1].reshape(E // SUBB, SUBB)
    srca = adjs[0].reshape(E // SUBBA, SUBBA)
    dsta = adjs[1].reshape(E // SUBBA, SUBBA)

    mask8 = jnp.asarray(_MASK8)
    A1 = jnp.concatenate([mask8 * a_src1.reshape(HC, 1),
                          mask8 * a_dst1.reshape(HC, 1)], axis=1)
    z7 = jnp.zeros((NCLS, H - 1), jnp.float32)
    A2 = jnp.concatenate([a_src2.reshape(NCLS, 1), z7,
                          a_dst2.reshape(NCLS, 1), z7], axis=1)

    z8 = jnp.zeros((N, H), jnp.float32)
    z64 = jnp.zeros((N, NCLS), jnp.float32)

    h1, al1 = _tc1(x, W1, A1)
    as1 = al1[:, :H]
    ad1 = al1[:, H:]

    dens, ex1 = _att(srca, dsta, as1, ad1, z8)
    osum = _msg1(src, dst, ex1, _tcd(dens), h1, z64)

    h2, al2 = _tc2(osum[0], osum[1], b1.reshape(1, HC), W2, A2)
    as2 = al2[:, :H]
    ad2 = al2[:, H:]

    dens2, ex2 = _att(srca, dsta, as2, ad2, z8)
    osum2 = _msg2(src, dst, ex2, _tcd(dens2), h2, z64)

    return _tc3(osum2[0], osum2[1], b2.reshape(1, NCLS))


# speedup vs baseline: 47.2644x; 1.0105x over previous
import numpy as np
import jax
import jax.numpy as jnp
from jax import lax
from jax.experimental import pallas as pl
from jax.experimental.pallas import tpu as pltpu
from jax.experimental.pallas import tpu_sc as plsc

N = 10000
E = 320000
D = 128
H = 8
CH = 8
HC = H * CH
NCLS = 64
SLOPE = 0.2
EPS = 1e-16

NC = 2
NS = 16
NW = NC * NS
EPW = E // NW
CB = 400
NCHUNK = EPW // CB
SUB = 10
SUBB = CB // SUB
HSUB = SUB // 2
HCB = CB // 2

CBA = 1000
NCHUNKA = EPW // CBA
SUBA = 10
SUBBA = CBA // SUBA
NPAIRA = (NCHUNKA + 1) // 2
NPAIR = (NCHUNK + 1) // 2

_MASK8 = np.zeros((HC, H), np.float32)
for _h in range(H):
    _MASK8[_h * CH:(_h + 1) * CH, _h] = 1.0


def _iota16():
    return lax.iota(jnp.int32, 16)


def _splat(v):
    return jnp.full((16,), v, jnp.int32)


def _leaky_exp(e):
    return jnp.exp(jnp.where(e > 0, e, SLOPE * e))


_SC_PARAMS = pltpu.CompilerParams(needs_layout_passes=False,
                                  use_tc_tiling_on_sc=False)


def _att_body(src_hbm, dst_hbm, as_hbm, ad_hbm, z8_hbm,
              den_out, ex_out,
              sA, dA, asrA, adrA, sB, dB, asrB, adrB,
              exc, accum, semA, semB):
    cid = lax.axis_index("c")
    sid = lax.axis_index("s")
    wid = sid * NC + cid
    base = wid * EPW
    rbase = base // SUBBA

    bufs = [(sA, dA, asrA, adrA, semA), (sB, dB, asrB, adrB, semB)]

    lane = _iota16()
    half = lane >> 3
    mod8 = lane & 7

    @pl.when(sid == 0)
    def _():
        pltpu.sync_copy(z8_hbm, accum)
    plsc.subcore_barrier()

    def fire(c, bi):
        s2, d2, asr, adr, sem = bufs[bi]
        roff = rbase + c * SUBA
        pltpu.sync_copy(src_hbm.at[pl.ds(roff, SUBA), :], s2)
        pltpu.sync_copy(dst_hbm.at[pl.ds(roff, SUBA), :], d2)
        for j in range(SUBA):
            isl = pl.ds(j * SUBBA, SUBBA)
            pltpu.async_copy(as_hbm.at[s2.at[j]], asr.at[isl, :], sem)
            pltpu.async_copy(ad_hbm.at[d2.at[j]], adr.at[isl, :], sem)

    def wait(bi):
        s2, d2, asr, adr, sem = bufs[bi]
        for j in range(SUBA):
            isl = pl.ds(j * SUBBA, SUBBA)
            pltpu.make_async_copy(as_hbm.at[s2.at[j]], asr.at[isl, :],
                                  sem).wait()
            pltpu.make_async_copy(ad_hbm.at[d2.at[j]], adr.at[isl, :],
                                  sem).wait()

    def compute(c, bi):
        s2, d2, asr, adr, sem = bufs[bi]

        def vec(k, _):
            for t in range(2):
                row = _splat(4 * k + 2 * t) + half
                e = (plsc.load_gather(asr, [row, mod8])
                     + plsc.load_gather(adr, [row, mod8]))
                plsc.store_scatter(exc, [row, mod8], _leaky_exp(e))
            return 0
        lax.fori_loop(0, CBA * H // 32, vec, 0)

        scps = []
        for j in range(SUBA):
            scps.append(pltpu.async_copy(
                exc.at[pl.ds(j * SUBBA, SUBBA), :],
                accum.at[d2.at[j]], sem, add=True))
        for cp in scps:
            cp.wait()
        pltpu.sync_copy(exc, ex_out.at[pl.ds(base + c * CBA, CBA), :])

    fire(0, 0)

    def pair(g, _):
        c0 = 2 * g
        c1 = 2 * g + 1
        c2 = 2 * g + 2

        @pl.when(c1 < NCHUNKA)
        def _():
            fire(c1, 1)
        wait(0)
        compute(c0, 0)

        @pl.when(c2 < NCHUNKA)
        def _():
            fire(c2, 0)

        @pl.when(c1 < NCHUNKA)
        def _():
            wait(1)
            compute(c1, 1)
        return 0

    lax.fori_loop(0, NPAIRA, pair, 0)
    plsc.subcore_barrier()

    @pl.when(sid == 0)
    def _():
        pltpu.sync_copy(accum, den_out.at[cid])


_att = pl.kernel(
    _att_body,
    out_type=(jax.ShapeDtypeStruct((NC, N, H), jnp.float32),
              jax.ShapeDtypeStruct((E, H), jnp.float32)),
    mesh=plsc.VectorSubcoreMesh(core_axis_name="c", subcore_axis_name="s"),
    scratch_types=[
        pltpu.VMEM((SUBA, SUBBA), jnp.int32),
        pltpu.VMEM((SUBA, SUBBA), jnp.int32),
        pltpu.VMEM((CBA, H), jnp.float32),
        pltpu.VMEM((CBA, H), jnp.float32),
        pltpu.VMEM((SUBA, SUBBA), jnp.int32),
        pltpu.VMEM((SUBA, SUBBA), jnp.int32),
        pltpu.VMEM((CBA, H), jnp.float32),
        pltpu.VMEM((CBA, H), jnp.float32),
        pltpu.VMEM((CBA, H), jnp.float32),
        pltpu.VMEM_SHARED((N, H), jnp.float32),
        pltpu.SemaphoreType.DMA,
        pltpu.SemaphoreType.DMA,
    ],
    compiler_params=_SC_PARAMS,
)


def _mk_msg(broadcast0):
    def body(src_hbm, dst_hbm, ex_hbm, den_hbm, h_hbm, z64_hbm,
             osum_out,
             sA, dA, excA, drA, hrA, sB, dB, excB, drB, hrB,
             msg, accum, semA, semB):
        cid = lax.axis_index("c")
        sid = lax.axis_index("s")
        wid = sid * NC + cid
        base = wid * EPW
        rbase = base // SUBB

        bufs = [(sA, dA, excA, drA, hrA, semA),
                (sB, dB, excB, drB, hrB, semB)]

        lane = _iota16()
        half = lane >> 3
        mod8 = lane & 7

        @pl.when(sid == 0)
        def _():
            pltpu.sync_copy(z64_hbm, accum)
        plsc.subcore_barrier()

        def fire(c, bi):
            s2, d2, exc, dr, hr, sem = bufs[bi]
            roff = rbase + c * SUB
            pltpu.sync_copy(src_hbm.at[pl.ds(roff, SUB), :], s2)
            pltpu.sync_copy(dst_hbm.at[pl.ds(roff, SUB), :], d2)
            pltpu.async_copy(ex_hbm.at[pl.ds(base + c * CB, CB), :], exc, sem)
            for j in range(SUB):
                isl = pl.ds(j * SUBB, SUBB)
                pltpu.async_copy(den_hbm.at[d2.at[j]], dr.at[isl, :], sem)
                pltpu.async_copy(h_hbm.at[s2.at[j]], hr.at[isl, :], sem)

        def wait(bi):
            s2, d2, exc, dr, hr, sem = bufs[bi]
            pltpu.make_async_copy(ex_hbm.at[pl.ds(0, CB), :], exc, sem).wait()
            for j in range(SUB):
                isl = pl.ds(j * SUBB, SUBB)
                pltpu.make_async_copy(den_hbm.at[d2.at[j]], dr.at[isl, :],
                                      sem).wait()
                pltpu.make_async_copy(h_hbm.at[s2.at[j]], hr.at[isl, :],
                                      sem).wait()

        acol = [_splat(2 * j) + half for j in range(4)]
        zcol = _splat(0)

        def compute(bi):
            s2, d2, exc, dr, hr, sem = bufs[bi]

            def veca(k, _):
                for t in range(2):
                    row = _splat(4 * k + 2 * t) + half
                    den = plsc.load_gather(dr, [row, mod8])
                    ex = plsc.load_gather(exc, [row, mod8])
                    plsc.store_scatter(exc, [row, mod8], ex / (den + EPS))
                return 0
            lax.fori_loop(0, CB * H // 32, veca, 0)

            for hb in range(2):
                eb = hb * HCB

                def vecb(i, _):
                    for t in range(4):
                        r = 4 * i + t
                        e = eb + r
                        if broadcast0:
                            av = plsc.load_gather(exc, [_splat(e), zcol])
                            for j in range(4):
                                sl = pl.ds(16 * j, 16)
                                msg[r, sl] = hr[e, sl] * av
                        else:
                            arow = _splat(e)
                            for j in range(4):
                                sl = pl.ds(16 * j, 16)
                                av = plsc.load_gather(exc, [arow, acol[j]])
                                msg[r, sl] = hr[e, sl] * av
                    return 0
                lax.fori_loop(0, HCB // 4, vecb, 0)

                scps = []
                for j in range(HSUB):
                    scps.append(pltpu.async_copy(
                        msg.at[pl.ds(j * SUBB, SUBB), :],
                        accum.at[d2.at[hb * HSUB + j]], sem, add=True))
                for cp in scps:
                    cp.wait()

        fire(0, 0)

        def pair(g, _):
            c1 = 2 * g + 1
            c2 = 2 * g + 2

            @pl.when(c1 < NCHUNK)
            def _():
                fire(c1, 1)
            wait(0)
            compute(0)

            @pl.when(c2 < NCHUNK)
            def _():
                fire(c2, 0)

            @pl.when(c1 < NCHUNK)
            def _():
                wait(1)
                compute(1)
            return 0

        lax.fori_loop(0, NPAIR, pair, 0)
        plsc.subcore_barrier()

        @pl.when(sid == 0)
        def _():
            pltpu.sync_copy(accum, osum_out.at[cid])

    return pl.kernel(
        body,
        out_type=jax.ShapeDtypeStruct((NC, N, HC), jnp.float32),
        mesh=plsc.VectorSubcoreMesh(core_axis_name="c", subcore_axis_name="s"),
        scratch_types=[
            pltpu.VMEM((SUB, SUBB), jnp.int32),
            pltpu.VMEM((SUB, SUBB), jnp.int32),
            pltpu.VMEM((CB, H), jnp.float32),
            pltpu.VMEM((CB, H), jnp.float32),
            pltpu.VMEM((CB, HC), jnp.float32),
            pltpu.VMEM((SUB, SUBB), jnp.int32),
            pltpu.VMEM((SUB, SUBB), jnp.int32),
            pltpu.VMEM((CB, H), jnp.float32),
            pltpu.VMEM((CB, H), jnp.float32),
            pltpu.VMEM((CB, HC), jnp.float32),
            pltpu.VMEM((HCB, HC), jnp.float32),
            pltpu.VMEM_SHARED((N, HC), jnp.float32),
            pltpu.SemaphoreType.DMA,
            pltpu.SemaphoreType.DMA,
        ],
        compiler_params=_SC_PARAMS,
    )


_msg1 = _mk_msg(False)
_msg2 = _mk_msg(True)


_RB = 1000
_PREC = jax.lax.Precision.HIGHEST


def _tc1_body(x_ref, w_ref, a_ref, h_ref, al_ref):
    h = lax.dot_general(x_ref[...], w_ref[...], (((1,), (0,)), ((), ())),
                        precision=_PREC, preferred_element_type=jnp.float32)
    h_ref[...] = h
    al_ref[...] = lax.dot_general(h, a_ref[...], (((1,), (0,)), ((), ())),
                                  precision=_PREC,
                                  preferred_element_type=jnp.float32)


_tc1 = pl.pallas_call(
    _tc1_body,
    grid=(N // _RB,),
    in_specs=[pl.BlockSpec((_RB, D), lambda i: (i, 0)),
              pl.BlockSpec((D, HC), lambda i: (0, 0)),
              pl.BlockSpec((HC, 2 * H), lambda i: (0, 0))],
    out_specs=[pl.BlockSpec((_RB, HC), lambda i: (i, 0)),
               pl.BlockSpec((_RB, 2 * H), lambda i: (i, 0))],
    out_shape=[jax.ShapeDtypeStruct((N, HC), jnp.float32),
               jax.ShapeDtypeStruct((N, 2 * H), jnp.float32)],
)


def _tc2_body(o0_ref, o1_ref, b_ref, w_ref, a_ref, h_ref, al_ref):
    g = o0_ref[...] + o1_ref[...] + b_ref[...]
    g = jnp.where(g > 0, g, jnp.exp(g) - 1.0)
    h = lax.dot_general(g, w_ref[...], (((1,), (0,)), ((), ())),
                        precision=_PREC, preferred_element_type=jnp.float32)
    h_ref[...] = h
    al_ref[...] = lax.dot_general(h, a_ref[...], (((1,), (0,)), ((), ())),
                                  precision=_PREC,
                                  preferred_element_type=jnp.float32)


_tc2 = pl.pallas_call(
    _tc2_body,
    grid=(N // _RB,),
    in_specs=[pl.BlockSpec((_RB, HC), lambda i: (i, 0)),
              pl.BlockSpec((_RB, HC), lambda i: (i, 0)),
              pl.BlockSpec((1, HC), lambda i: (0, 0)),
              pl.BlockSpec((HC, NCLS), lambda i: (0, 0)),
              pl.BlockSpec((NCLS, 2 * H), lambda i: (0, 0))],
    out_specs=[pl.BlockSpec((_RB, NCLS), lambda i: (i, 0)),
               pl.BlockSpec((_RB, 2 * H), lambda i: (i, 0))],
    out_shape=[jax.ShapeDtypeStruct((N, NCLS), jnp.float32),
               jax.ShapeDtypeStruct((N, 2 * H), jnp.float32)],
)


def _tcd_body(d_ref, o_ref):
    o_ref[...] = d_ref[0] + d_ref[1]


_tcd = pl.pallas_call(
    _tcd_body,
    grid=(N // _RB,),
    in_specs=[pl.BlockSpec((NC, _RB, H), lambda i: (0, i, 0))],
    out_specs=pl.BlockSpec((_RB, H), lambda i: (i, 0)),
    out_shape=jax.ShapeDtypeStruct((N, H), jnp.float32),
)


def _tc3_body(s0_ref, s1_ref, b_ref, o_ref):
    o_ref[...] = s0_ref[...] + s1_ref[...] + b_ref[...]


_tc3 = pl.pallas_call(
    _tc3_body,
    grid=(N // _RB,),
    in_specs=[pl.BlockSpec((_RB, NCLS), lambda i: (i, 0)),
              pl.BlockSpec((_RB, NCLS), lambda i: (i, 0)),
              pl.BlockSpec((1, NCLS), lambda i: (0, 0))],
    out_specs=pl.BlockSpec((_RB, NCLS), lambda i: (i, 0)),
    out_shape=jax.ShapeDtypeStruct((N, NCLS), jnp.float32),
)


def kernel(x, adjs, W1, a_src1, a_dst1, b1, W2, a_src2, a_dst2, b2):
    adjs = adjs.astype(jnp.int32)
    src = adjs[0].reshape(E // SUBB, SUBB)
    dst = adjs[1].reshape(E // SUBB, SUBB)
    srca = adjs[0].reshape(E // SUBBA, SUBBA)
    dsta = adjs[1].reshape(E // SUBBA, SUBBA)

    mask8 = jnp.asarray(_MASK8)
    A1 = jnp.concatenate([mask8 * a_src1.reshape(HC, 1),
                          mask8 * a_dst1.reshape(HC, 1)], axis=1)
    z7 = jnp.zeros((NCLS, H - 1), jnp.float32)
    A2 = jnp.concatenate([a_src2.reshape(NCLS, 1), z7,
                          a_dst2.reshape(NCLS, 1), z7], axis=1)

    z8 = jnp.zeros((N, H), jnp.float32)
    z64 = jnp.zeros((N, NCLS), jnp.float32)

    h1, al1 = _tc1(x, W1, A1)
    as1 = al1[:, :H]
    ad1 = al1[:, H:]

    dens, ex1 = _att(srca, dsta, as1, ad1, z8)
    osum = _msg1(src, dst, ex1, _tcd(dens), h1, z64)

    h2, al2 = _tc2(osum[0], osum[1], b1.reshape(1, HC), W2, A2)
    as2 = al2[:, :H]
    ad2 = al2[:, H:]

    dens2, ex2 = _att(srca, dsta, as2, ad2, z8)
    osum2 = _msg2(src, dst, ex2, _tcd(dens2), h2, z64)

    return _tc3(osum2[0], osum2[1], b2.reshape(1, NCLS))


# speedup vs baseline: 49.2835x; 1.0427x over previous
import numpy as np
import jax
import jax.numpy as jnp
from jax import lax
from jax.experimental import pallas as pl
from jax.experimental.pallas import tpu as pltpu
from jax.experimental.pallas import tpu_sc as plsc

N = 10000
E = 320000
D = 128
H = 8
CH = 8
HC = H * CH
NCLS = 64
SLOPE = 0.2
EPS = 1e-16

NC = 2
NS = 16
NW = NC * NS
EPW = E // NW
CB = 400
NCHUNK = EPW // CB
SUB = 10
SUBB = CB // SUB
HSUB = SUB // 2
HCB = CB // 2

CBA = 1000
NCHUNKA = EPW // CBA
SUBA = 10
SUBBA = CBA // SUBA
NPAIRA = (NCHUNKA + 1) // 2
NPAIR = (NCHUNK + 1) // 2

_MASK8 = np.zeros((HC, H), np.float32)
for _h in range(H):
    _MASK8[_h * CH:(_h + 1) * CH, _h] = 1.0


def _iota16():
    return lax.iota(jnp.int32, 16)


def _splat(v):
    return jnp.full((16,), v, jnp.int32)


def _leaky_exp(e):
    return jnp.exp(jnp.where(e > 0, e, SLOPE * e))


_SC_PARAMS = pltpu.CompilerParams(needs_layout_passes=False,
                                  use_tc_tiling_on_sc=False)


def _att_body(src_hbm, dst_hbm, as_hbm, ad_hbm, z8_hbm,
              den_out, ex_out,
              sA, dA, asrA, adrA, sB, dB, asrB, adrB,
              exc, accum, semA, semB):
    cid = lax.axis_index("c")
    sid = lax.axis_index("s")
    wid = sid * NC + cid
    base = wid * EPW
    rbase = base // SUBBA

    bufs = [(sA, dA, asrA, adrA, semA), (sB, dB, asrB, adrB, semB)]

    lane = _iota16()
    half = lane >> 3
    mod8 = lane & 7

    @pl.when(sid == 0)
    def _():
        pltpu.sync_copy(z8_hbm, accum)
    plsc.subcore_barrier()

    def fire(c, bi):
        s2, d2, asr, adr, sem = bufs[bi]
        roff = rbase + c * SUBA
        pltpu.sync_copy(src_hbm.at[pl.ds(roff, SUBA), :], s2)
        pltpu.sync_copy(dst_hbm.at[pl.ds(roff, SUBA), :], d2)
        for j in range(SUBA):
            isl = pl.ds(j * SUBBA, SUBBA)
            pltpu.async_copy(as_hbm.at[s2.at[j]], asr.at[isl, :], sem)
            pltpu.async_copy(ad_hbm.at[d2.at[j]], adr.at[isl, :], sem)

    def wait(bi):
        s2, d2, asr, adr, sem = bufs[bi]
        for j in range(SUBA):
            isl = pl.ds(j * SUBBA, SUBBA)
            pltpu.make_async_copy(as_hbm.at[s2.at[j]], asr.at[isl, :],
                                  sem).wait()
            pltpu.make_async_copy(ad_hbm.at[d2.at[j]], adr.at[isl, :],
                                  sem).wait()

    def compute(c, bi):
        s2, d2, asr, adr, sem = bufs[bi]

        def vec(k, _):
            for t in range(2):
                row = _splat(4 * k + 2 * t) + half
                e = (plsc.load_gather(asr, [row, mod8])
                     + plsc.load_gather(adr, [row, mod8]))
                plsc.store_scatter(exc, [row, mod8], _leaky_exp(e))
            return 0
        lax.fori_loop(0, CBA * H // 32, vec, 0)

        scps = []
        for j in range(SUBA):
            scps.append(pltpu.async_copy(
                exc.at[pl.ds(j * SUBBA, SUBBA), :],
                accum.at[d2.at[j]], sem, add=True))
        for cp in scps:
            cp.wait()
        pltpu.sync_copy(exc, ex_out.at[pl.ds(base + c * CBA, CBA), :])

    fire(0, 0)

    def pair(g, _):
        c0 = 2 * g
        c1 = 2 * g + 1
        c2 = 2 * g + 2

        @pl.when(c1 < NCHUNKA)
        def _():
            fire(c1, 1)
        wait(0)
        compute(c0, 0)

        @pl.when(c2 < NCHUNKA)
        def _():
            fire(c2, 0)

        @pl.when(c1 < NCHUNKA)
        def _():
            wait(1)
            compute(c1, 1)
        return 0

    lax.fori_loop(0, NPAIRA, pair, 0)
    plsc.subcore_barrier()

    @pl.when(sid == 0)
    def _():
        pltpu.sync_copy(accum, den_out.at[cid])


_att = pl.kernel(
    _att_body,
    out_type=(jax.ShapeDtypeStruct((NC, N, H), jnp.float32),
              jax.ShapeDtypeStruct((E, H), jnp.float32)),
    mesh=plsc.VectorSubcoreMesh(core_axis_name="c", subcore_axis_name="s"),
    scratch_types=[
        pltpu.VMEM((SUBA, SUBBA), jnp.int32),
        pltpu.VMEM((SUBA, SUBBA), jnp.int32),
        pltpu.VMEM((CBA, H), jnp.float32),
        pltpu.VMEM((CBA, H), jnp.float32),
        pltpu.VMEM((SUBA, SUBBA), jnp.int32),
        pltpu.VMEM((SUBA, SUBBA), jnp.int32),
        pltpu.VMEM((CBA, H), jnp.float32),
        pltpu.VMEM((CBA, H), jnp.float32),
        pltpu.VMEM((CBA, H), jnp.float32),
        pltpu.VMEM_SHARED((N, H), jnp.float32),
        pltpu.SemaphoreType.DMA,
        pltpu.SemaphoreType.DMA,
    ],
    compiler_params=_SC_PARAMS,
)


def _mk_msg(broadcast0):
    def body(src_hbm, dst_hbm, ex_hbm, den_hbm, h_hbm, z64_hbm,
             osum_out,
             sidx, dA, excA, drA, hrA, dB, excB, drB, hrB,
             msg0, msg1, accum, semA, semB, semS0, semS1):
        cid = lax.axis_index("c")
        sid = lax.axis_index("s")
        wid = sid * NC + cid
        base = wid * EPW
        rbase = base // SUBB

        bufs = [(dA, excA, drA, hrA, semA),
                (dB, excB, drB, hrB, semB)]
        msgs = [msg0, msg1]
        ssems = [semS0, semS1]

        lane = _iota16()
        half = lane >> 3
        mod8 = lane & 7

        @pl.when(sid == 0)
        def _():
            pltpu.sync_copy(z64_hbm, accum)
        plsc.subcore_barrier()

        def fire(c, bi):
            d2, exc, dr, hr, sem = bufs[bi]
            roff = rbase + c * SUB
            pltpu.sync_copy(src_hbm.at[pl.ds(roff, SUB), :], sidx)
            pltpu.sync_copy(dst_hbm.at[pl.ds(roff, SUB), :], d2)
            pltpu.async_copy(ex_hbm.at[pl.ds(base + c * CB, CB), :], exc, sem)
            for j in range(SUB):
                isl = pl.ds(j * SUBB, SUBB)
                pltpu.async_copy(den_hbm.at[d2.at[j]], dr.at[isl, :], sem)
                pltpu.async_copy(h_hbm.at[sidx.at[j]], hr.at[isl, :], sem)

        def wait(bi):
            d2, exc, dr, hr, sem = bufs[bi]
            pltpu.make_async_copy(ex_hbm.at[pl.ds(0, CB), :], exc, sem).wait()
            for j in range(SUB):
                isl = pl.ds(j * SUBB, SUBB)
                pltpu.make_async_copy(den_hbm.at[d2.at[j]], dr.at[isl, :],
                                      sem).wait()
                pltpu.make_async_copy(h_hbm.at[sidx.at[j]], hr.at[isl, :],
                                      sem).wait()

        acol = [_splat(2 * j) + half for j in range(4)]
        zcol = _splat(0)

        def drain_msg(hb):
            for j in range(HSUB):
                pltpu.make_async_copy(
                    z64_hbm.at[pl.ds(0, SUBB)],
                    msgs[hb].at[pl.ds(j * SUBB, SUBB), :], ssems[hb]).wait()

        def compute(bi, drain):
            d2, exc, dr, hr, sem = bufs[bi]

            def veca(k, _):
                for t in range(2):
                    row = _splat(4 * k + 2 * t) + half
                    den = plsc.load_gather(dr, [row, mod8])
                    ex = plsc.load_gather(exc, [row, mod8])
                    plsc.store_scatter(exc, [row, mod8], ex / (den + EPS))
                return 0
            lax.fori_loop(0, CB * H // 32, veca, 0)

            for hb in range(2):
                eb = hb * HCB
                drain(hb)

                def vecb(i, _):
                    for t in range(4):
                        r = 4 * i + t
                        e = eb + r
                        if broadcast0:
                            av = plsc.load_gather(exc, [_splat(e), zcol])
                            for j in range(4):
                                sl = pl.ds(16 * j, 16)
                                msgs[hb][r, sl] = hr[e, sl] * av
                        else:
                            arow = _splat(e)
                            for j in range(4):
                                sl = pl.ds(16 * j, 16)
                                av = plsc.load_gather(exc, [arow, acol[j]])
                                msgs[hb][r, sl] = hr[e, sl] * av
                    return 0
                lax.fori_loop(0, HCB // 4, vecb, 0)

                for j in range(HSUB):
                    pltpu.async_copy(
                        msgs[hb].at[pl.ds(j * SUBB, SUBB), :],
                        accum.at[d2.at[hb * HSUB + j]], ssems[hb], add=True)

        fire(0, 0)

        def pair(g, _):
            c1 = 2 * g + 1
            c2 = 2 * g + 2

            wait(0)

            @pl.when(c1 < NCHUNK)
            def _():
                fire(c1, 1)

            def drain0(hb):
                @pl.when(g > 0)
                def _():
                    drain_msg(hb)
            compute(0, drain0)

            @pl.when(c1 < NCHUNK)
            def _():
                wait(1)

            @pl.when(c2 < NCHUNK)
            def _():
                fire(c2, 0)

            @pl.when(c1 < NCHUNK)
            def _():
                compute(1, drain_msg)
            return 0

        lax.fori_loop(0, NPAIR, pair, 0)
        drain_msg(0)
        drain_msg(1)
        plsc.subcore_barrier()

        @pl.when(sid == 0)
        def _():
            pltpu.sync_copy(accum, osum_out.at[cid])

    return pl.kernel(
        body,
        out_type=jax.ShapeDtypeStruct((NC, N, HC), jnp.float32),
        mesh=plsc.VectorSubcoreMesh(core_axis_name="c", subcore_axis_name="s"),
        scratch_types=[
            pltpu.VMEM((SUB, SUBB), jnp.int32),
            pltpu.VMEM((SUB, SUBB), jnp.int32),
            pltpu.VMEM((CB, H), jnp.float32),
            pltpu.VMEM((CB, H), jnp.float32),
            pltpu.VMEM((CB, HC), jnp.float32),
            pltpu.VMEM((SUB, SUBB), jnp.int32),
            pltpu.VMEM((CB, H), jnp.float32),
            pltpu.VMEM((CB, H), jnp.float32),
            pltpu.VMEM((CB, HC), jnp.float32),
            pltpu.VMEM((HCB, HC), jnp.float32),
            pltpu.VMEM((HCB, HC), jnp.float32),
            pltpu.VMEM_SHARED((N, HC), jnp.float32),
            pltpu.SemaphoreType.DMA,
            pltpu.SemaphoreType.DMA,
            pltpu.SemaphoreType.DMA,
            pltpu.SemaphoreType.DMA,
        ],
        compiler_params=_SC_PARAMS,
    )


_msg1 = _mk_msg(False)
_msg2 = _mk_msg(True)


_RB = 1000
_PREC = jax.lax.Precision.HIGHEST


def _tc1_body(x_ref, w_ref, a_ref, h_ref, al_ref):
    h = lax.dot_general(x_ref[...], w_ref[...], (((1,), (0,)), ((), ())),
                        precision=_PREC, preferred_element_type=jnp.float32)
    h_ref[...] = h
    al_ref[...] = lax.dot_general(h, a_ref[...], (((1,), (0,)), ((), ())),
                                  precision=_PREC,
                                  preferred_element_type=jnp.float32)


_tc1 = pl.pallas_call(
    _tc1_body,
    grid=(N // _RB,),
    in_specs=[pl.BlockSpec((_RB, D), lambda i: (i, 0)),
              pl.BlockSpec((D, HC), lambda i: (0, 0)),
              pl.BlockSpec((HC, 2 * H), lambda i: (0, 0))],
    out_specs=[pl.BlockSpec((_RB, HC), lambda i: (i, 0)),
               pl.BlockSpec((_RB, 2 * H), lambda i: (i, 0))],
    out_shape=[jax.ShapeDtypeStruct((N, HC), jnp.float32),
               jax.ShapeDtypeStruct((N, 2 * H), jnp.float32)],
)


def _tc2_body(o0_ref, o1_ref, b_ref, w_ref, a_ref, h_ref, al_ref):
    g = o0_ref[...] + o1_ref[...] + b_ref[...]
    g = jnp.where(g > 0, g, jnp.exp(g) - 1.0)
    h = lax.dot_general(g, w_ref[...], (((1,), (0,)), ((), ())),
                        precision=_PREC, preferred_element_type=jnp.float32)
    h_ref[...] = h
    al_ref[...] = lax.dot_general(h, a_ref[...], (((1,), (0,)), ((), ())),
                                  precision=_PREC,
                                  preferred_element_type=jnp.float32)


_tc2 = pl.pallas_call(
    _tc2_body,
    grid=(N // _RB,),
    in_specs=[pl.BlockSpec((_RB, HC), lambda i: (i, 0)),
              pl.BlockSpec((_RB, HC), lambda i: (i, 0)),
              pl.BlockSpec((1, HC), lambda i: (0, 0)),
              pl.BlockSpec((HC, NCLS), lambda i: (0, 0)),
              pl.BlockSpec((NCLS, 2 * H), lambda i: (0, 0))],
    out_specs=[pl.BlockSpec((_RB, NCLS), lambda i: (i, 0)),
               pl.BlockSpec((_RB, 2 * H), lambda i: (i, 0))],
    out_shape=[jax.ShapeDtypeStruct((N, NCLS), jnp.float32),
               jax.ShapeDtypeStruct((N, 2 * H), jnp.float32)],
)


def _tcd_body(d_ref, o_ref):
    o_ref[...] = d_ref[0] + d_ref[1]


_tcd = pl.pallas_call(
    _tcd_body,
    grid=(N // _RB,),
    in_specs=[pl.BlockSpec((NC, _RB, H), lambda i: (0, i, 0))],
    out_specs=pl.BlockSpec((_RB, H), lambda i: (i, 0)),
    out_shape=jax.ShapeDtypeStruct((N, H), jnp.float32),
)


def _tc3_body(s0_ref, s1_ref, b_ref, o_ref):
    o_ref[...] = s0_ref[...] + s1_ref[...] + b_ref[...]


_tc3 = pl.pallas_call(
    _tc3_body,
    grid=(N // _RB,),
    in_specs=[pl.BlockSpec((_RB, NCLS), lambda i: (i, 0)),
              pl.BlockSpec((_RB, NCLS), lambda i: (i, 0)),
              pl.BlockSpec((1, NCLS), lambda i: (0, 0))],
    out_specs=pl.BlockSpec((_RB, NCLS), lambda i: (i, 0)),
    out_shape=jax.ShapeDtypeStruct((N, NCLS), jnp.float32),
)


def kernel(x, adjs, W1, a_src1, a_dst1, b1, W2, a_src2, a_dst2, b2):
    adjs = adjs.astype(jnp.int32)
    src = adjs[0].reshape(E // SUBB, SUBB)
    dst = adjs[1].reshape(E // SUBB, SUBB)
    srca = adjs[0].reshape(E // SUBBA, SUBBA)
    dsta = adjs[1].reshape(E // SUBBA, SUBBA)

    mask8 = jnp.asarray(_MASK8)
    A1 = jnp.concatenate([mask8 * a_src1.reshape(HC, 1),
                          mask8 * a_dst1.reshape(HC, 1)], axis=1)
    z7 = jnp.zeros((NCLS, H - 1), jnp.float32)
    A2 = jnp.concatenate([a_src2.reshape(NCLS, 1), z7,
                          a_dst2.reshape(NCLS, 1), z7], axis=1)

    z8 = jnp.zeros((N, H), jnp.float32)
    z64 = jnp.zeros((N, NCLS), jnp.float32)

    h1, al1 = _tc1(x, W1, A1)
    as1 = al1[:, :H]
    ad1 = al1[:, H:]

    dens, ex1 = _att(srca, dsta, as1, ad1, z8)
    osum = _msg1(src, dst, ex1, _tcd(dens), h1, z64)

    h2, al2 = _tc2(osum[0], osum[1], b1.reshape(1, HC), W2, A2)
    as2 = al2[:, :H]
    ad2 = al2[:, H:]

    dens2, ex2 = _att(srca, dsta, as2, ad2, z8)
    osum2 = _msg2(src, dst, ex2, _tcd(dens2), h2, z64)

    return _tc3(osum2[0], osum2[1], b2.reshape(1, NCLS))
